# baseline (speedup 1.0000x reference)
import jax
import jax.numpy as jnp
from jax.experimental import pallas as pl

N = 100000
G = 256
H = 64


def _gat_layer(x, W, a_src, a_dst, b, src, dst):
    h = x @ W
    alpha_s = h @ a_src
    alpha_d = h @ a_dst
    e = alpha_s[src] + alpha_d[dst]
    e = jax.nn.leaky_relu(e, negative_slope=0.2)
    e_max = jax.ops.segment_max(e, dst, num_segments=N)
    e = jnp.exp(e - e_max[dst])
    denom = jax.ops.segment_sum(e, dst, num_segments=N)
    alpha = e / denom[dst]
    out = jax.ops.segment_sum(alpha[:, None] * h[src], dst, num_segments=N)
    return out + b


def _head_kernel(p_ref, w_ref, b_ref, o_ref):
    o_ref[...] = p_ref[...] @ w_ref[...] + b_ref[...]


def kernel(x, edge_index, batch, W1, a_src1, a_dst1, b1, W2, a_src2, a_dst2, b2, W3, a_src3, a_dst3, b3, Wp, bp):
    loop = jnp.arange(N, dtype=edge_index.dtype)
    src = jnp.concatenate([edge_index[0], loop])
    dst = jnp.concatenate([edge_index[1], loop])

    h = _gat_layer(x, W1, a_src1, a_dst1, b1, src, dst)
    h = jax.nn.relu(h)
    h = _gat_layer(h, W2, a_src2, a_dst2, b2, src, dst)
    h = jax.nn.relu(h)
    h = _gat_layer(h, W3, a_src3, a_dst3, b3, src, dst)
    h = jax.nn.relu(h)

    sums = jax.ops.segment_sum(h, batch, num_segments=G)
    counts = jax.ops.segment_sum(jnp.ones((N,), dtype=jnp.float32), batch, num_segments=G)
    pooled = sums / jnp.maximum(counts, 1.0)[:, None]

    Wp_pad = jnp.zeros((H, 128), jnp.float32).at[:, :5].set(Wp)
    bp_pad = jnp.zeros((128,), jnp.float32).at[:5].set(bp)
    out_pad = pl.pallas_call(
        _head_kernel,
        out_shape=jax.ShapeDtypeStruct((G, 128), jnp.float32),
    )(pooled, Wp_pad, jnp.broadcast_to(bp_pad, (G, 128)))
    return out_pad[:, :5]


# speedup vs baseline: 10.1601x; 10.1601x over previous
import functools

import jax
import jax.numpy as jnp
from jax import lax
from jax.experimental import pallas as pl
from jax.experimental.pallas import tpu as pltpu
from jax.experimental.pallas import tpu_sc as plsc

N = 100000
E = 1600000
G = 256
H = 64
NUM_OUT = 5

K = 512
ET = 53248
WN = ET // K
EP = 32 * ET
NF = H // 16
ND = N + 16

R = 2048
NB = 49
NPAD = NB * R

PW = 3136
PWIN = 112
PNW = PW // PWIN
PG = 272


def _iota16():
    return lax.iota(jnp.int32, 16)


def _dense_body(first, refs):
    i = pl.program_id(0)
    if first:
        (x_ref, d0_ref, d1_ref, bp_ref, w_ref, asrc_ref, adst_ref,
         h_ref, as_ref, ad_ref, mx_ref) = refs
        act = x_ref[...]
    else:
        (a00, a01, a02, a03, a10, a11, a12, a13, d0_ref, d1_ref, bp_ref,
         w_ref, asrc_ref, adst_ref, h_ref, as_ref, ad_ref, mx_ref) = refs
        num = jnp.concatenate(
            [a00[...] + a10[...], a01[...] + a11[...],
             a02[...] + a12[...], a03[...] + a13[...]], axis=1)
        den = (d0_ref[...] + d1_ref[...]).reshape(R, 1)
        act = jnp.maximum(num / den + bp_ref[...], 0.0)
    h = jnp.dot(act, w_ref[...], preferred_element_type=jnp.float32)
    h_ref[...] = h
    asb = jnp.dot(h, asrc_ref[...], preferred_element_type=jnp.float32)[:, 0]
    adb = jnp.dot(h, adst_ref[...], preferred_element_type=jnp.float32)[:, 0]
    as_ref[...] = asb.reshape(1, 1, R)
    ad_ref[...] = adb.reshape(1, 1, R)
    valid = i * R + lax.iota(jnp.int32, R) < N
    bmax = jnp.max(jnp.where(valid, asb, -3.4e38))

    @pl.when(i == 0)
    def _():
        mx_ref[...] = jnp.full((1, 1), -3.4e38, jnp.float32)
    mx_ref[...] = jnp.maximum(mx_ref[...], bmax)


def _dense_layer(first, acc_parts, den_pair, b_prev, W, a_src, a_dst):
    kin = acc_parts[0].shape[1] if first else H
    grid = (NB,)
    out_shapes = (
        jax.ShapeDtypeStruct((N, H), jnp.float32),
        jax.ShapeDtypeStruct((NB, 1, R), jnp.float32),
        jax.ShapeDtypeStruct((NB, 1, R), jnp.float32),
        jax.ShapeDtypeStruct((1, 1), jnp.float32),
    )
    part_specs = ([pl.BlockSpec((R, kin), lambda i: (i, 0))] if first else
                  [pl.BlockSpec((R, 16), lambda i: (i, 0))] * 8)
    in_specs = part_specs + [
        pl.BlockSpec((1, 1, R), lambda i: (i, 0, 0)),
        pl.BlockSpec((1, 1, R), lambda i: (i, 0, 0)),
        pl.BlockSpec((1, H), lambda i: (0, 0)),
        pl.BlockSpec((kin, H), lambda i: (0, 0)),
        pl.BlockSpec((H, 1), lambda i: (0, 0)),
        pl.BlockSpec((H, 1), lambda i: (0, 0)),
    ]
    out_specs = (
        pl.BlockSpec((R, H), lambda i: (i, 0)),
        pl.BlockSpec((1, 1, R), lambda i: (i, 0, 0)),
        pl.BlockSpec((1, 1, R), lambda i: (i, 0, 0)),
        pl.BlockSpec((1, 1), lambda i: (0, 0)),
    )

    def body(*refs):
        _dense_body(first, refs)

    h, as3d, ad3d, mx = pl.pallas_call(
        body,
        grid=grid,
        in_specs=in_specs,
        out_specs=out_specs,
        out_shape=out_shapes,
    )(*acc_parts, den_pair[0], den_pair[1], b_prev.reshape(1, H), W,
      a_src.reshape(H, 1), a_dst.reshape(H, 1))
    as_flat = as3d.reshape(NPAD)[:N]
    ad_flat = ad3d.reshape(NPAD)[:N]
    mx16 = jnp.broadcast_to(mx.reshape(1), (16,))
    return h, as_flat, ad_flat, mx16


def _edge_body(hcat_hbm, as_hbm, ad_hbm, mx_hbm, src_hbm, dst_hbm,
               z16_hbm, z1_hbm,
               acc_hbm, den_hbm,
               srcw, dstw, asv, adv, wbuf, gix, dli, hsl, mxv, acc_sh,
               den_sh, sem):
    c = lax.axis_index("c")
    s = lax.axis_index("s")
    w32 = c * 16 + s
    base_e = w32 * ET
    pltpu.sync_copy(mx_hbm, mxv)
    asmax = mxv[pl.ds(0, 16)][0]

    def pass_body(p, _):
        @pl.when(s < 10)
        def _():
            pltpu.sync_copy(z16_hbm, acc_sh.at[pl.ds(s * 10000, 10000), :])

        @pl.when(s == 10)
        def _():
            pltpu.sync_copy(z16_hbm.at[pl.ds(0, 16), :],
                            acc_sh.at[pl.ds(N, 16), :])

        @pl.when(p == 0)
        def _():
            @pl.when(s < 10)
            def _():
                pltpu.sync_copy(z1_hbm, den_sh.at[pl.ds(s * 10000, 10000)])

            @pl.when(s == 10)
            def _():
                pltpu.sync_copy(z1_hbm.at[pl.ds(0, 16)],
                                den_sh.at[pl.ds(N, 16)])
        plsc.subcore_barrier()

        def window_body(wi, _):
            eb = base_e + wi * K
            pltpu.sync_copy(src_hbm.at[pl.ds(eb, K)], srcw)
            pltpu.sync_copy(dst_hbm.at[pl.ds(eb, K)], dstw)
            for t in range(K // 16):
                sl = pl.ds((t % 8) * 16, 16)
                gix[t // 8, sl] = srcw[pl.ds(t * 16, 16)] + p * N
                dli[t // 8, sl] = dstw[pl.ds(t * 16, 16)]
            descs = []
            for j in range(4):
                descs.append(pltpu.async_copy(
                    hcat_hbm.at[gix.at[j]],
                    hsl.at[pl.ds(j * 128, 128), :], sem))
            for j in range(4):
                pltpu.sync_copy(as_hbm.at[srcw.at[pl.ds(j * 128, 128)]],
                                asv.at[pl.ds(j * 128, 128)])
                pltpu.sync_copy(ad_hbm.at[dstw.at[pl.ds(j * 128, 128)]],
                                adv.at[pl.ds(j * 128, 128)])
            for t in range(K // 16):
                sl = pl.ds(t * 16, 16)
                av = asv[sl]
                bv = adv[sl]
                z = av + bv
                e = jnp.maximum(z, 0.2 * z)
                u = asmax + bv
                cb = jnp.maximum(u, 0.2 * u)
                wbuf[sl] = jnp.exp(e - cb)
            for d in descs:
                d.wait()

            def scale_grp(gb, _):
                wv = wbuf[pl.ds(gb * 16, 16)]
                for i in range(16):
                    r = gb * 16 + i
                    hsl[r, pl.ds(0, 16)] = hsl[r, pl.ds(0, 16)] * wv[i]
                return 0
            lax.fori_loop(0, K // 16, scale_grp, 0)

            for j in range(4):
                pltpu.sync_copy(hsl.at[pl.ds(j * 128, 128), :],
                                acc_sh.at[dli.at[j]], add=True)

            @pl.when(p == 0)
            def _():
                for j in range(4):
                    pltpu.sync_copy(wbuf.at[pl.ds(j * 128, 128)],
                                    den_sh.at[dli.at[j]], add=True)
            return 0

        lax.fori_loop(0, WN, window_body, 0)
        plsc.subcore_barrier()

        @pl.when(s < 10)
        def _():
            pltpu.sync_copy(
                acc_sh.at[pl.ds(s * 10000, 10000), :],
                acc_hbm.at[pl.ds((c * NF + p) * N + s * 10000, 10000), :])

            @pl.when(p == 0)
            def _():
                pltpu.sync_copy(
                    den_sh.at[pl.ds(s * 10000, 10000)],
                    den_hbm.at[pl.ds(c * N + s * 10000, 10000)])
        plsc.subcore_barrier()
        return 0

    lax.fori_loop(0, NF, pass_body, 0)


def _make_edge_kernel():
    mesh = plsc.VectorSubcoreMesh(core_axis_name="c", subcore_axis_name="s")
    return pl.kernel(
        _edge_body,
        out_type=(
            jax.ShapeDtypeStruct((2 * NF * N, 16), jnp.float32),
            jax.ShapeDtypeStruct((2 * N,), jnp.float32),
        ),
        mesh=mesh,
        compiler_params=pltpu.CompilerParams(use_tc_tiling_on_sc=False),
        scratch_types=[
            pltpu.VMEM((K,), jnp.int32),
            pltpu.VMEM((K,), jnp.int32),
            pltpu.VMEM((K,), jnp.float32),
            pltpu.VMEM((K,), jnp.float32),
            pltpu.VMEM((K,), jnp.float32),
            pltpu.VMEM((4, 128), jnp.int32),
            pltpu.VMEM((4, 128), jnp.int32),
            pltpu.VMEM((K, 16), jnp.float32),
            pltpu.VMEM((16,), jnp.float32),
            pltpu.VMEM_SHARED((ND, 16), jnp.float32),
            pltpu.VMEM_SHARED((ND,), jnp.float32),
            pltpu.SemaphoreType.DMA,
        ],
    )


def _pool_body(a0_hbm, a1_hbm, d_hbm, b_hbm, batch_hbm, zp_hbm, zc_hbm,
               psum_hbm, pcnt_hbm,
               arow, brow, drow, bidx, ones_v, bvec, psh, csh):
    c = lax.axis_index("c")
    s = lax.axis_index("s")
    w = s * 2 + c
    pltpu.sync_copy(b_hbm, bvec)
    bch = [bvec[pl.ds(j * 16, 16)] for j in range(NF)]
    for t in range(PWIN // 16):
        ones_v[pl.ds(t * 16, 16)] = jnp.ones((16,), jnp.float32)

    @pl.when(s < 2)
    def _():
        pltpu.sync_copy(zp_hbm, psh.at[pl.ds(s * (PG // 2), PG // 2), :])
        pltpu.sync_copy(zc_hbm, csh.at[pl.ds(s * (PG // 2), PG // 2)])
    plsc.subcore_barrier()

    def win_body(wi, _):
        nb = w * PW + wi * PWIN
        for j in range(NF):
            pltpu.sync_copy(a0_hbm.at[pl.ds(j * NPAD + nb, PWIN), :],
                            arow.at[pl.ds(0, PWIN), pl.ds(j * 16, 16)])
            pltpu.sync_copy(a1_hbm.at[pl.ds(j * NPAD + nb, PWIN), :],
                            brow.at[pl.ds(0, PWIN), pl.ds(j * 16, 16)])
        pltpu.sync_copy(d_hbm.at[pl.ds(nb, PWIN)], drow.at[pl.ds(0, PWIN)])
        pltpu.sync_copy(d_hbm.at[pl.ds(NPAD + nb, PWIN)],
                        drow.at[pl.ds(PWIN, PWIN)])
        pltpu.sync_copy(batch_hbm.at[pl.ds(nb, PWIN)], bidx.at[0])

        def row_grp(t, _):
            dv = (drow[pl.ds(t * 16, 16)] +
                  drow[pl.ds(PWIN + t * 16, 16)])
            inv = 1.0 / dv
            for i in range(16):
                r = t * 16 + i
                for j in range(NF):
                    sl = pl.ds(j * 16, 16)
                    v = arow[r, sl] + brow[r, sl]
                    arow[r, sl] = jnp.maximum(v * inv[i] + bch[j], 0.0)
            return 0
        lax.fori_loop(0, PWIN // 16, row_grp, 0)

        pltpu.sync_copy(arow, psh.at[bidx.at[0]], add=True)
        pltpu.sync_copy(ones_v, csh.at[bidx.at[0]], add=True)
        return 0
    lax.fori_loop(0, PNW, win_body, 0)
    plsc.subcore_barrier()

    @pl.when(s < 2)
    def _():
        pltpu.sync_copy(psh.at[pl.ds(s * (PG // 2), PG // 2), :],
                        psum_hbm.at[pl.ds(c * PG + s * (PG // 2), PG // 2), :])
        pltpu.sync_copy(csh.at[pl.ds(s * (PG // 2), PG // 2)],
                        pcnt_hbm.at[pl.ds(c * PG + s * (PG // 2), PG // 2)])


def _make_pool_kernel():
    mesh = plsc.VectorSubcoreMesh(core_axis_name="c", subcore_axis_name="s")
    return pl.kernel(
        _pool_body,
        out_type=(
            jax.ShapeDtypeStruct((2 * PG, H), jnp.float32),
            jax.ShapeDtypeStruct((2 * PG,), jnp.float32),
        ),
        mesh=mesh,
        compiler_params=pltpu.CompilerParams(use_tc_tiling_on_sc=False),
        scratch_types=[
            pltpu.VMEM((PWIN, H), jnp.float32),
            pltpu.VMEM((PWIN, H), jnp.float32),
            pltpu.VMEM((2 * PWIN,), jnp.float32),
            pltpu.VMEM((1, PWIN), jnp.int32),
            pltpu.VMEM((PWIN,), jnp.float32),
            pltpu.VMEM((H,), jnp.float32),
            pltpu.VMEM_SHARED((PG, H), jnp.float32),
            pltpu.VMEM_SHARED((PG,), jnp.float32),
        ],
    )


def _head_body(ps_ref, pc_ref, wp_ref, bp_ref, o_ref):
    sums = ps_ref[0, :G, :] + ps_ref[1, :G, :]
    cnts = pc_ref[0, 0, :G] + pc_ref[1, 0, :G]
    pooled = sums / jnp.maximum(cnts, 1.0)[:, None]
    o_ref[...] = jnp.dot(pooled, wp_ref[...],
                         preferred_element_type=jnp.float32) + bp_ref[...]


def _split_parts(accp, denp):
    parts = [accp[(c * NF + j) * N:(c * NF + j + 1) * N]
             for c in range(2) for j in range(NF)]
    den0 = jnp.pad(denp[:N], (0, NPAD - N),
                   constant_values=1.0).reshape(NB, 1, R)
    den1 = jnp.pad(denp[N:], (0, NPAD - N),
                   constant_values=1.0).reshape(NB, 1, R)
    return parts, (den0, den1)


def kernel(x, edge_index, batch, W1, a_src1, a_dst1, b1, W2, a_src2, a_dst2,
           b2, W3, a_src3, a_dst3, b3, Wp, bp):
    f32 = jnp.float32
    loop = jnp.arange(N, dtype=edge_index.dtype)
    pad_e = EP - (E + N)
    srcp = jnp.concatenate([edge_index[0], loop,
                            jnp.zeros((pad_e,), jnp.int32)])
    dstp = jnp.concatenate([edge_index[1], loop,
                            jnp.full((pad_e,), N, jnp.int32)])

    x_pad = jnp.pad(x, ((0, 0), (0, 5)))
    W1p = jnp.pad(W1, ((0, 5), (0, 0)))

    z16 = jnp.zeros((10000, 16), f32)
    z1 = jnp.zeros((10000,), f32)
    dummy_den = jnp.ones((NB, 1, R), f32)

    edge_k = _make_edge_kernel()

    def run_layer(first, parts, dens, b_prev, W, a_src, a_dst):
        h, as_, ad_, mx = _dense_layer(first, parts, dens, b_prev, W,
                                       a_src, a_dst)
        hcat = jnp.concatenate([h[:, j * 16:(j + 1) * 16]
                                for j in range(NF)], axis=0)
        ad_pad = jnp.pad(ad_, (0, 16))
        accp, denp = edge_k(hcat, as_, ad_pad, mx, srcp, dstp, z16, z1)
        return _split_parts(accp, denp)

    parts1, dens1 = run_layer(True, [x_pad], (dummy_den, dummy_den),
                              b1, W1p, a_src1, a_dst1)
    parts2, dens2 = run_layer(False, parts1, dens1, b1, W2, a_src2, a_dst2)
    parts3, dens3 = run_layer(False, parts2, dens2, b2, W3, a_src3, a_dst3)

    a0 = jnp.concatenate(parts3[:NF], axis=0)
    a1 = jnp.concatenate(parts3[NF:], axis=0)
    a0 = jnp.pad(a0.reshape(NF, N, 16), ((0, 0), (0, NPAD - N), (0, 0))
                 ).reshape(NF * NPAD, 16)
    a1 = jnp.pad(a1.reshape(NF, N, 16), ((0, 0), (0, NPAD - N), (0, 0))
                 ).reshape(NF * NPAD, 16)
    dnp = jnp.concatenate([
        jnp.pad(dens3[0].reshape(NPAD)[:N], (0, NPAD - N), constant_values=1.0),
        jnp.pad(dens3[1].reshape(NPAD)[:N], (0, NPAD - N), constant_values=1.0),
    ])
    batchp = jnp.pad(batch, (0, NPAD - N), constant_values=G)
    zp = jnp.zeros((PG // 2, H), f32)
    zc = jnp.zeros((PG // 2,), f32)
    pool_k = _make_pool_kernel()
    psum, pcnt = pool_k(a0, a1, dnp, b3, batchp, zp, zc)

    Wpp = jnp.zeros((H, 128), f32).at[:, :NUM_OUT].set(Wp)
    bpp = jnp.zeros((1, 128), f32).at[0, :NUM_OUT].set(bp)
    out_pad = pl.pallas_call(
        _head_body,
        out_shape=jax.ShapeDtypeStruct((G, 128), f32),
    )(psum.reshape(2, PG, H), pcnt.reshape(2, 1, PG), Wpp, bpp)
    return out_pad[:, :NUM_OUT]


# speedup vs baseline: 17.1416x; 1.6872x over previous
import functools

import jax
import jax.numpy as jnp
from jax import lax
from jax.experimental import pallas as pl
from jax.experimental.pallas import tpu as pltpu
from jax.experimental.pallas import tpu_sc as plsc

N = 100000
E = 1600000
G = 256
H = 64
NUM_OUT = 5

K = 512
ET = 53248
WN = ET // K
EP = 32 * ET
NF = H // 16
ND = N + 16

R = 2048
NB = 49
NPAD = NB * R

PW = 3136
PWIN = 112
PNW = PW // PWIN
PG = 272


def _iota16():
    return lax.iota(jnp.int32, 16)


def _dense_body(first, refs):
    i = pl.program_id(0)
    if first:
        (x_ref, d0_ref, d1_ref, bp_ref, w_ref, asrc_ref, adst_ref,
         h_ref, as_ref, ad_ref, mx_ref) = refs
        act = x_ref[...]
    else:
        (a00, a01, a02, a03, a10, a11, a12, a13, d0_ref, d1_ref, bp_ref,
         w_ref, asrc_ref, adst_ref, h_ref, as_ref, ad_ref, mx_ref) = refs
        num = jnp.concatenate(
            [a00[...] + a10[...], a01[...] + a11[...],
             a02[...] + a12[...], a03[...] + a13[...]], axis=1)
        den = (d0_ref[...] + d1_ref[...]).reshape(R, 1)
        act = jnp.maximum(num / den + bp_ref[...], 0.0)
    h = jnp.dot(act, w_ref[...], preferred_element_type=jnp.float32)
    h_ref[...] = h
    asb = jnp.dot(h, asrc_ref[...], preferred_element_type=jnp.float32)[:, 0]
    adb = jnp.dot(h, adst_ref[...], preferred_element_type=jnp.float32)[:, 0]
    as_ref[...] = asb.reshape(1, 1, R)
    ad_ref[...] = adb.reshape(1, 1, R)
    valid = i * R + lax.iota(jnp.int32, R) < N
    bmax = jnp.max(jnp.where(valid, asb, -3.4e38))

    @pl.when(i == 0)
    def _():
        mx_ref[...] = jnp.full((1, 1), -3.4e38, jnp.float32)
    mx_ref[...] = jnp.maximum(mx_ref[...], bmax)


def _dense_layer(first, acc_parts, den_pair, b_prev, W, a_src, a_dst):
    kin = acc_parts[0].shape[1] if first else H
    grid = (NB,)
    out_shapes = (
        jax.ShapeDtypeStruct((N, H), jnp.float32),
        jax.ShapeDtypeStruct((NB, 1, R), jnp.float32),
        jax.ShapeDtypeStruct((NB, 1, R), jnp.float32),
        jax.ShapeDtypeStruct((1, 1), jnp.float32),
    )
    part_specs = ([pl.BlockSpec((R, kin), lambda i: (i, 0))] if first else
                  [pl.BlockSpec((R, 16), lambda i: (i, 0))] * 8)
    in_specs = part_specs + [
        pl.BlockSpec((1, 1, R), lambda i: (i, 0, 0)),
        pl.BlockSpec((1, 1, R), lambda i: (i, 0, 0)),
        pl.BlockSpec((1, H), lambda i: (0, 0)),
        pl.BlockSpec((kin, H), lambda i: (0, 0)),
        pl.BlockSpec((H, 1), lambda i: (0, 0)),
        pl.BlockSpec((H, 1), lambda i: (0, 0)),
    ]
    out_specs = (
        pl.BlockSpec((R, H), lambda i: (i, 0)),
        pl.BlockSpec((1, 1, R), lambda i: (i, 0, 0)),
        pl.BlockSpec((1, 1, R), lambda i: (i, 0, 0)),
        pl.BlockSpec((1, 1), lambda i: (0, 0)),
    )

    def body(*refs):
        _dense_body(first, refs)

    h, as3d, ad3d, mx = pl.pallas_call(
        body,
        grid=grid,
        in_specs=in_specs,
        out_specs=out_specs,
        out_shape=out_shapes,
    )(*acc_parts, den_pair[0], den_pair[1], b_prev.reshape(1, H), W,
      a_src.reshape(H, 1), a_dst.reshape(H, 1))
    as_flat = as3d.reshape(NPAD)[:N]
    ad_flat = ad3d.reshape(NPAD)[:N]
    mx16 = jnp.broadcast_to(mx.reshape(1), (16,))
    return h, as_flat, ad_flat, mx16


def _edge_body(hcat_hbm, as_hbm, ad_hbm, mx_hbm, src_hbm, dst_hbm,
               z16_hbm, z1_hbm,
               acc_hbm, den_hbm,
               srcw, dstw, asv, adv, wbuf, gix, dli, hsl, mxv, acc_sh,
               den_sh, sem, sem2):
    c = lax.axis_index("c")
    s = lax.axis_index("s")
    w32 = c * 16 + s
    base_e = w32 * ET
    pltpu.sync_copy(mx_hbm, mxv)
    asmax = mxv[pl.ds(0, 16)][0]

    def pass_body(p, _):
        @pl.when(s < 10)
        def _():
            pltpu.sync_copy(z16_hbm, acc_sh.at[pl.ds(s * 10000, 10000), :])

        @pl.when(s == 10)
        def _():
            pltpu.sync_copy(z16_hbm.at[pl.ds(0, 16), :],
                            acc_sh.at[pl.ds(N, 16), :])

        @pl.when(p == 0)
        def _():
            @pl.when(s < 10)
            def _():
                pltpu.sync_copy(z1_hbm, den_sh.at[pl.ds(s * 10000, 10000)])

            @pl.when(s == 10)
            def _():
                pltpu.sync_copy(z1_hbm.at[pl.ds(0, 16)],
                                den_sh.at[pl.ds(N, 16)])
        plsc.subcore_barrier()

        def window_body(wi, _):
            eb = base_e + wi * K
            d1 = pltpu.async_copy(src_hbm.at[pl.ds(eb, K)], srcw, sem)
            d2 = pltpu.async_copy(dst_hbm.at[pl.ds(eb, K)], dstw, sem)
            d1.wait()
            d2.wait()
            for t in range(K // 16):
                sl = pl.ds((t % 8) * 16, 16)
                gix[t // 8, sl] = srcw[pl.ds(t * 16, 16)] + p * N
                dli[t // 8, sl] = dstw[pl.ds(t * 16, 16)]
            descs = []
            for j in range(4):
                descs.append(pltpu.async_copy(
                    hcat_hbm.at[gix.at[j]],
                    hsl.at[pl.ds(j * 128, 128), :], sem))
            for j in range(4):
                descs.append(pltpu.async_copy(
                    as_hbm.at[srcw.at[pl.ds(j * 128, 128)]],
                    asv.at[pl.ds(j * 128, 128)], sem))
                descs.append(pltpu.async_copy(
                    ad_hbm.at[dstw.at[pl.ds(j * 128, 128)]],
                    adv.at[pl.ds(j * 128, 128)], sem))
            for d in descs:
                d.wait()
            for t in range(K // 16):
                sl = pl.ds(t * 16, 16)
                av = asv[sl]
                bv = adv[sl]
                z = av + bv
                e = jnp.maximum(z, 0.2 * z)
                u = asmax + bv
                cb = jnp.maximum(u, 0.2 * u)
                wbuf[sl] = jnp.exp(e - cb)

            def scale_grp(gb, _):
                wv = wbuf[pl.ds(gb * 16, 16)]
                for i in range(16):
                    r = gb * 16 + i
                    hsl[r, pl.ds(0, 16)] = hsl[r, pl.ds(0, 16)] * wv[i]
                return 0
            lax.fori_loop(0, K // 16, scale_grp, 0)

            sdescs = []
            for j in range(4):
                sdescs.append(pltpu.async_copy(
                    hsl.at[pl.ds(j * 128, 128), :],
                    acc_sh.at[dli.at[j]], sem2, add=True))
            for d in sdescs:
                d.wait()

            @pl.when(p == 0)
            def _():
                ddescs = []
                for j in range(4):
                    ddescs.append(pltpu.async_copy(
                        wbuf.at[pl.ds(j * 128, 128)],
                        den_sh.at[dli.at[j]], sem2, add=True))
                for d in ddescs:
                    d.wait()
            return 0

        lax.fori_loop(0, WN, window_body, 0)
        plsc.subcore_barrier()

        @pl.when(s < 10)
        def _():
            pltpu.sync_copy(
                acc_sh.at[pl.ds(s * 10000, 10000), :],
                acc_hbm.at[pl.ds((c * NF + p) * N + s * 10000, 10000), :])

            @pl.when(p == 0)
            def _():
                pltpu.sync_copy(
                    den_sh.at[pl.ds(s * 10000, 10000)],
                    den_hbm.at[pl.ds(c * N + s * 10000, 10000)])
        plsc.subcore_barrier()
        return 0

    lax.fori_loop(0, NF, pass_body, 0)


def _make_edge_kernel():
    mesh = plsc.VectorSubcoreMesh(core_axis_name="c", subcore_axis_name="s")
    return pl.kernel(
        _edge_body,
        out_type=(
            jax.ShapeDtypeStruct((2 * NF * N, 16), jnp.float32),
            jax.ShapeDtypeStruct((2 * N,), jnp.float32),
        ),
        mesh=mesh,
        compiler_params=pltpu.CompilerParams(use_tc_tiling_on_sc=False),
        scratch_types=[
            pltpu.VMEM((K,), jnp.int32),
            pltpu.VMEM((K,), jnp.int32),
            pltpu.VMEM((K,), jnp.float32),
            pltpu.VMEM((K,), jnp.float32),
            pltpu.VMEM((K,), jnp.float32),
            pltpu.VMEM((4, 128), jnp.int32),
            pltpu.VMEM((4, 128), jnp.int32),
            pltpu.VMEM((K, 16), jnp.float32),
            pltpu.VMEM((16,), jnp.float32),
            pltpu.VMEM_SHARED((ND, 16), jnp.float32),
            pltpu.VMEM_SHARED((ND,), jnp.float32),
            pltpu.SemaphoreType.DMA,
            pltpu.SemaphoreType.DMA,
        ],
    )


def _pool_body(a0_hbm, a1_hbm, d_hbm, b_hbm, batch_hbm, zp_hbm, zc_hbm,
               psum_hbm, pcnt_hbm,
               arow, brow, drow, bidx, ones_v, bvec, psh, csh):
    c = lax.axis_index("c")
    s = lax.axis_index("s")
    w = s * 2 + c
    pltpu.sync_copy(b_hbm, bvec)
    bch = [bvec[pl.ds(j * 16, 16)] for j in range(NF)]
    for t in range(PWIN // 16):
        ones_v[pl.ds(t * 16, 16)] = jnp.ones((16,), jnp.float32)

    @pl.when(s < 2)
    def _():
        pltpu.sync_copy(zp_hbm, psh.at[pl.ds(s * (PG // 2), PG // 2), :])
        pltpu.sync_copy(zc_hbm, csh.at[pl.ds(s * (PG // 2), PG // 2)])
    plsc.subcore_barrier()

    def win_body(wi, _):
        nb = w * PW + wi * PWIN
        for j in range(NF):
            pltpu.sync_copy(a0_hbm.at[pl.ds(j * NPAD + nb, PWIN), :],
                            arow.at[pl.ds(0, PWIN), pl.ds(j * 16, 16)])
            pltpu.sync_copy(a1_hbm.at[pl.ds(j * NPAD + nb, PWIN), :],
                            brow.at[pl.ds(0, PWIN), pl.ds(j * 16, 16)])
        pltpu.sync_copy(d_hbm.at[pl.ds(nb, PWIN)], drow.at[pl.ds(0, PWIN)])
        pltpu.sync_copy(d_hbm.at[pl.ds(NPAD + nb, PWIN)],
                        drow.at[pl.ds(PWIN, PWIN)])
        pltpu.sync_copy(batch_hbm.at[pl.ds(nb, PWIN)], bidx.at[0])

        def row_grp(t, _):
            dv = (drow[pl.ds(t * 16, 16)] +
                  drow[pl.ds(PWIN + t * 16, 16)])
            inv = 1.0 / dv
            for i in range(16):
                r = t * 16 + i
                for j in range(NF):
                    sl = pl.ds(j * 16, 16)
                    v = arow[r, sl] + brow[r, sl]
                    arow[r, sl] = jnp.maximum(v * inv[i] + bch[j], 0.0)
            return 0
        lax.fori_loop(0, PWIN // 16, row_grp, 0)

        pltpu.sync_copy(arow, psh.at[bidx.at[0]], add=True)
        pltpu.sync_copy(ones_v, csh.at[bidx.at[0]], add=True)
        return 0
    lax.fori_loop(0, PNW, win_body, 0)
    plsc.subcore_barrier()

    @pl.when(s < 2)
    def _():
        pltpu.sync_copy(psh.at[pl.ds(s * (PG // 2), PG // 2), :],
                        psum_hbm.at[pl.ds(c * PG + s * (PG // 2), PG // 2), :])
        pltpu.sync_copy(csh.at[pl.ds(s * (PG // 2), PG // 2)],
                        pcnt_hbm.at[pl.ds(c * PG + s * (PG // 2), PG // 2)])


def _make_pool_kernel():
    mesh = plsc.VectorSubcoreMesh(core_axis_name="c", subcore_axis_name="s")
    return pl.kernel(
        _pool_body,
        out_type=(
            jax.ShapeDtypeStruct((2 * PG, H), jnp.float32),
            jax.ShapeDtypeStruct((2 * PG,), jnp.float32),
        ),
        mesh=mesh,
        compiler_params=pltpu.CompilerParams(use_tc_tiling_on_sc=False),
        scratch_types=[
            pltpu.VMEM((PWIN, H), jnp.float32),
            pltpu.VMEM((PWIN, H), jnp.float32),
            pltpu.VMEM((2 * PWIN,), jnp.float32),
            pltpu.VMEM((1, PWIN), jnp.int32),
            pltpu.VMEM((PWIN,), jnp.float32),
            pltpu.VMEM((H,), jnp.float32),
            pltpu.VMEM_SHARED((PG, H), jnp.float32),
            pltpu.VMEM_SHARED((PG,), jnp.float32),
        ],
    )


def _head_body(ps_ref, pc_ref, wp_ref, bp_ref, o_ref):
    sums = ps_ref[0, :G, :] + ps_ref[1, :G, :]
    cnts = pc_ref[0, 0, :G] + pc_ref[1, 0, :G]
    pooled = sums / jnp.maximum(cnts, 1.0)[:, None]
    o_ref[...] = jnp.dot(pooled, wp_ref[...],
                         preferred_element_type=jnp.float32) + bp_ref[...]


def _split_parts(accp, denp):
    parts = [accp[(c * NF + j) * N:(c * NF + j + 1) * N]
             for c in range(2) for j in range(NF)]
    den0 = jnp.pad(denp[:N], (0, NPAD - N),
                   constant_values=1.0).reshape(NB, 1, R)
    den1 = jnp.pad(denp[N:], (0, NPAD - N),
                   constant_values=1.0).reshape(NB, 1, R)
    return parts, (den0, den1)


def kernel(x, edge_index, batch, W1, a_src1, a_dst1, b1, W2, a_src2, a_dst2,
           b2, W3, a_src3, a_dst3, b3, Wp, bp):
    f32 = jnp.float32
    loop = jnp.arange(N, dtype=edge_index.dtype)
    pad_e = EP - (E + N)
    srcp = jnp.concatenate([edge_index[0], loop,
                            jnp.zeros((pad_e,), jnp.int32)])
    dstp = jnp.concatenate([edge_index[1], loop,
                            jnp.full((pad_e,), N, jnp.int32)])

    x_pad = jnp.pad(x, ((0, 0), (0, 5)))
    W1p = jnp.pad(W1, ((0, 5), (0, 0)))

    z16 = jnp.zeros((10000, 16), f32)
    z1 = jnp.zeros((10000,), f32)
    dummy_den = jnp.ones((NB, 1, R), f32)

    edge_k = _make_edge_kernel()

    def run_layer(first, parts, dens, b_prev, W, a_src, a_dst):
        h, as_, ad_, mx = _dense_layer(first, parts, dens, b_prev, W,
                                       a_src, a_dst)
        hcat = jnp.concatenate([h[:, j * 16:(j + 1) * 16]
                                for j in range(NF)], axis=0)
        ad_pad = jnp.pad(ad_, (0, 16))
        accp, denp = edge_k(hcat, as_, ad_pad, mx, srcp, dstp, z16, z1)
        return _split_parts(accp, denp)

    parts1, dens1 = run_layer(True, [x_pad], (dummy_den, dummy_den),
                              b1, W1p, a_src1, a_dst1)
    parts2, dens2 = run_layer(False, parts1, dens1, b1, W2, a_src2, a_dst2)
    parts3, dens3 = run_layer(False, parts2, dens2, b2, W3, a_src3, a_dst3)

    a0 = jnp.concatenate(parts3[:NF], axis=0)
    a1 = jnp.concatenate(parts3[NF:], axis=0)
    a0 = jnp.pad(a0.reshape(NF, N, 16), ((0, 0), (0, NPAD - N), (0, 0))
                 ).reshape(NF * NPAD, 16)
    a1 = jnp.pad(a1.reshape(NF, N, 16), ((0, 0), (0, NPAD - N), (0, 0))
                 ).reshape(NF * NPAD, 16)
    dnp = jnp.concatenate([
        jnp.pad(dens3[0].reshape(NPAD)[:N], (0, NPAD - N), constant_values=1.0),
        jnp.pad(dens3[1].reshape(NPAD)[:N], (0, NPAD - N), constant_values=1.0),
    ])
    batchp = jnp.pad(batch, (0, NPAD - N), constant_values=G)
    zp = jnp.zeros((PG // 2, H), f32)
    zc = jnp.zeros((PG // 2,), f32)
    pool_k = _make_pool_kernel()
    psum, pcnt = pool_k(a0, a1, dnp, b3, batchp, zp, zc)

    Wpp = jnp.zeros((H, 128), f32).at[:, :NUM_OUT].set(Wp)
    bpp = jnp.zeros((1, 128), f32).at[0, :NUM_OUT].set(bp)
    out_pad = pl.pallas_call(
        _head_body,
        out_shape=jax.ShapeDtypeStruct((G, 128), f32),
    )(psum.reshape(2, PG, H), pcnt.reshape(2, 1, PG), Wpp, bpp)
    return out_pad[:, :NUM_OUT]


# speedup vs baseline: 22.1092x; 1.2898x over previous
import functools

import jax
import jax.numpy as jnp
from jax import lax
from jax.experimental import pallas as pl
from jax.experimental.pallas import tpu as pltpu
from jax.experimental.pallas import tpu_sc as plsc

N = 100000
E = 1600000
G = 256
H = 64
NUM_OUT = 5

K = 1024
ET = 53248
WN = ET // K
EP = 32 * ET
NF = H // 16
ND = N + 16

R = 2048
NB = 49
NPAD = NB * R

PW = 3136
PWIN = 112
PNW = PW // PWIN
PG = 272


def _iota16():
    return lax.iota(jnp.int32, 16)


def _dense_body(first, refs):
    i = pl.program_id(0)
    if first:
        (x_ref, d0_ref, d1_ref, bp_ref, w_ref, asrc_ref, adst_ref,
         h_ref, as_ref, ad_ref, mx_ref) = refs
        act = x_ref[...]
    else:
        (a00, a01, a02, a03, a10, a11, a12, a13, d0_ref, d1_ref, bp_ref,
         w_ref, asrc_ref, adst_ref, h_ref, as_ref, ad_ref, mx_ref) = refs
        num = jnp.concatenate(
            [a00[...] + a10[...], a01[...] + a11[...],
             a02[...] + a12[...], a03[...] + a13[...]], axis=1)
        den = (d0_ref[...] + d1_ref[...]).reshape(R, 1)
        act = jnp.maximum(num / den + bp_ref[...], 0.0)
    h = jnp.dot(act, w_ref[...], preferred_element_type=jnp.float32)
    h_ref[...] = h
    asb = jnp.dot(h, asrc_ref[...], preferred_element_type=jnp.float32)[:, 0]
    adb = jnp.dot(h, adst_ref[...], preferred_element_type=jnp.float32)[:, 0]
    as_ref[...] = asb.reshape(1, 1, R)
    ad_ref[...] = adb.reshape(1, 1, R)
    valid = i * R + lax.iota(jnp.int32, R) < N
    bmax = jnp.max(jnp.where(valid, asb, -3.4e38))

    @pl.when(i == 0)
    def _():
        mx_ref[...] = jnp.full((1, 1), -3.4e38, jnp.float32)
    mx_ref[...] = jnp.maximum(mx_ref[...], bmax)


def _dense_layer(first, acc_parts, den_pair, b_prev, W, a_src, a_dst):
    kin = acc_parts[0].shape[1] if first else H
    grid = (NB,)
    out_shapes = (
        jax.ShapeDtypeStruct((N, H), jnp.float32),
        jax.ShapeDtypeStruct((NB, 1, R), jnp.float32),
        jax.ShapeDtypeStruct((NB, 1, R), jnp.float32),
        jax.ShapeDtypeStruct((1, 1), jnp.float32),
    )
    part_specs = ([pl.BlockSpec((R, kin), lambda i: (i, 0))] if first else
                  [pl.BlockSpec((R, 16), lambda i: (i, 0))] * 8)
    in_specs = part_specs + [
        pl.BlockSpec((1, 1, R), lambda i: (i, 0, 0)),
        pl.BlockSpec((1, 1, R), lambda i: (i, 0, 0)),
        pl.BlockSpec((1, H), lambda i: (0, 0)),
        pl.BlockSpec((kin, H), lambda i: (0, 0)),
        pl.BlockSpec((H, 1), lambda i: (0, 0)),
        pl.BlockSpec((H, 1), lambda i: (0, 0)),
    ]
    out_specs = (
        pl.BlockSpec((R, H), lambda i: (i, 0)),
        pl.BlockSpec((1, 1, R), lambda i: (i, 0, 0)),
        pl.BlockSpec((1, 1, R), lambda i: (i, 0, 0)),
        pl.BlockSpec((1, 1), lambda i: (0, 0)),
    )

    def body(*refs):
        _dense_body(first, refs)

    h, as3d, ad3d, mx = pl.pallas_call(
        body,
        grid=grid,
        in_specs=in_specs,
        out_specs=out_specs,
        out_shape=out_shapes,
    )(*acc_parts, den_pair[0], den_pair[1], b_prev.reshape(1, H), W,
      a_src.reshape(H, 1), a_dst.reshape(H, 1))
    as_flat = as3d.reshape(NPAD)[:N]
    ad_flat = ad3d.reshape(NPAD)[:N]
    mx16 = jnp.broadcast_to(mx.reshape(1), (16,))
    return h, as_flat, ad_flat, mx16


def _edge_body(hcat_hbm, as_hbm, ad_hbm, mx_hbm, src_hbm, dst_hbm,
               z16_hbm, z1_hbm,
               acc_hbm, den_hbm, w_hbm,
               srcw, dstw, asv, adv, wbuf, gix, dli, hsl, mxv, acc_sh,
               den_sh, sem, sem2):
    c = lax.axis_index("c")
    s = lax.axis_index("s")
    w32 = c * 16 + s
    base_e = w32 * ET
    pltpu.sync_copy(mx_hbm, mxv)
    asmax = mxv[pl.ds(0, 16)][0]

    def pass_body(p, _):
        @pl.when(s < 10)
        def _():
            pltpu.sync_copy(z16_hbm, acc_sh.at[pl.ds(s * 10000, 10000), :])

        @pl.when(s == 10)
        def _():
            pltpu.sync_copy(z16_hbm.at[pl.ds(0, 16), :],
                            acc_sh.at[pl.ds(N, 16), :])

        @pl.when(p == 0)
        def _():
            @pl.when(s < 10)
            def _():
                pltpu.sync_copy(z1_hbm, den_sh.at[pl.ds(s * 10000, 10000)])

            @pl.when(s == 10)
            def _():
                pltpu.sync_copy(z1_hbm.at[pl.ds(0, 16)],
                                den_sh.at[pl.ds(N, 16)])
        plsc.subcore_barrier()

        def window_body(wi, _):
            eb = base_e + wi * K
            d1 = pltpu.async_copy(src_hbm.at[pl.ds(eb, K)], srcw, sem)
            d2 = pltpu.async_copy(dst_hbm.at[pl.ds(eb, K)], dstw, sem)
            d1.wait()
            d2.wait()
            for t in range(K // 16):
                sl = pl.ds((t % 8) * 16, 16)
                gix[t // 8, sl] = srcw[pl.ds(t * 16, 16)] + p * N
                dli[t // 8, sl] = dstw[pl.ds(t * 16, 16)]
            descs = []
            for j in range(K // 128):
                descs.append(pltpu.async_copy(
                    hcat_hbm.at[gix.at[j]],
                    hsl.at[pl.ds(j * 128, 128), :], sem))

            @pl.when(p == 0)
            def _():
                adescs = []
                for j in range(K // 128):
                    adescs.append(pltpu.async_copy(
                        as_hbm.at[srcw.at[pl.ds(j * 128, 128)]],
                        asv.at[pl.ds(j * 128, 128)], sem))
                    adescs.append(pltpu.async_copy(
                        ad_hbm.at[dstw.at[pl.ds(j * 128, 128)]],
                        adv.at[pl.ds(j * 128, 128)], sem))
                for d in adescs:
                    d.wait()
                for t in range(K // 16):
                    sl = pl.ds(t * 16, 16)
                    av = asv[sl]
                    bv = adv[sl]
                    z = av + bv
                    e = jnp.maximum(z, 0.2 * z)
                    u = asmax + bv
                    cb = jnp.maximum(u, 0.2 * u)
                    wbuf[sl] = jnp.exp(e - cb)
                pltpu.async_copy(wbuf, w_hbm.at[pl.ds(eb, K)], sem).wait()

            @pl.when(p != 0)
            def _():
                pltpu.async_copy(w_hbm.at[pl.ds(eb, K)], wbuf, sem).wait()
            for d in descs:
                d.wait()

            def scale_grp(gb, _):
                wv = wbuf[pl.ds(gb * 16, 16)]
                for i in range(16):
                    r = gb * 16 + i
                    hsl[r, pl.ds(0, 16)] = hsl[r, pl.ds(0, 16)] * wv[i]
                return 0
            lax.fori_loop(0, K // 16, scale_grp, 0)

            sdescs = []
            for j in range(K // 128):
                sdescs.append(pltpu.async_copy(
                    hsl.at[pl.ds(j * 128, 128), :],
                    acc_sh.at[dli.at[j]], sem2, add=True))

            @pl.when(p == 0)
            def _():
                ddescs = []
                for j in range(K // 128):
                    ddescs.append(pltpu.async_copy(
                        wbuf.at[pl.ds(j * 128, 128)],
                        den_sh.at[dli.at[j]], sem2, add=True))
                for d in ddescs:
                    d.wait()
            for d in sdescs:
                d.wait()
            return 0

        lax.fori_loop(0, WN, window_body, 0)
        plsc.subcore_barrier()

        @pl.when(s < 10)
        def _():
            pltpu.sync_copy(
                acc_sh.at[pl.ds(s * 10000, 10000), :],
                acc_hbm.at[pl.ds((c * NF + p) * N + s * 10000, 10000), :])

            @pl.when(p == 0)
            def _():
                pltpu.sync_copy(
                    den_sh.at[pl.ds(s * 10000, 10000)],
                    den_hbm.at[pl.ds(c * N + s * 10000, 10000)])
        plsc.subcore_barrier()
        return 0

    lax.fori_loop(0, NF, pass_body, 0)


def _make_edge_kernel():
    mesh = plsc.VectorSubcoreMesh(core_axis_name="c", subcore_axis_name="s")
    return pl.kernel(
        _edge_body,
        out_type=(
            jax.ShapeDtypeStruct((2 * NF * N, 16), jnp.float32),
            jax.ShapeDtypeStruct((2 * N,), jnp.float32),
            jax.ShapeDtypeStruct((EP,), jnp.float32),
        ),
        mesh=mesh,
        compiler_params=pltpu.CompilerParams(use_tc_tiling_on_sc=False),
        scratch_types=[
            pltpu.VMEM((K,), jnp.int32),
            pltpu.VMEM((K,), jnp.int32),
            pltpu.VMEM((K,), jnp.float32),
            pltpu.VMEM((K,), jnp.float32),
            pltpu.VMEM((K,), jnp.float32),
            pltpu.VMEM((K // 128, 128), jnp.int32),
            pltpu.VMEM((K // 128, 128), jnp.int32),
            pltpu.VMEM((K, 16), jnp.float32),
            pltpu.VMEM((16,), jnp.float32),
            pltpu.VMEM_SHARED((ND, 16), jnp.float32),
            pltpu.VMEM_SHARED((ND,), jnp.float32),
            pltpu.SemaphoreType.DMA,
            pltpu.SemaphoreType.DMA,
        ],
    )


def _pool_body(a0_hbm, a1_hbm, d_hbm, b_hbm, batch_hbm, zp_hbm, zc_hbm,
               psum_hbm, pcnt_hbm,
               arow, brow, drow, bidx, ones_v, bvec, psh, csh):
    c = lax.axis_index("c")
    s = lax.axis_index("s")
    w = s * 2 + c
    pltpu.sync_copy(b_hbm, bvec)
    bch = [bvec[pl.ds(j * 16, 16)] for j in range(NF)]
    for t in range(PWIN // 16):
        ones_v[pl.ds(t * 16, 16)] = jnp.ones((16,), jnp.float32)

    @pl.when(s < 2)
    def _():
        pltpu.sync_copy(zp_hbm, psh.at[pl.ds(s * (PG // 2), PG // 2), :])
        pltpu.sync_copy(zc_hbm, csh.at[pl.ds(s * (PG // 2), PG // 2)])
    plsc.subcore_barrier()

    def win_body(wi, _):
        nb = w * PW + wi * PWIN
        for j in range(NF):
            pltpu.sync_copy(a0_hbm.at[pl.ds(j * NPAD + nb, PWIN), :],
                            arow.at[pl.ds(0, PWIN), pl.ds(j * 16, 16)])
            pltpu.sync_copy(a1_hbm.at[pl.ds(j * NPAD + nb, PWIN), :],
                            brow.at[pl.ds(0, PWIN), pl.ds(j * 16, 16)])
        pltpu.sync_copy(d_hbm.at[pl.ds(nb, PWIN)], drow.at[pl.ds(0, PWIN)])
        pltpu.sync_copy(d_hbm.at[pl.ds(NPAD + nb, PWIN)],
                        drow.at[pl.ds(PWIN, PWIN)])
        pltpu.sync_copy(batch_hbm.at[pl.ds(nb, PWIN)], bidx.at[0])

        def row_grp(t, _):
            dv = (drow[pl.ds(t * 16, 16)] +
                  drow[pl.ds(PWIN + t * 16, 16)])
            inv = 1.0 / dv
            for i in range(16):
                r = t * 16 + i
                for j in range(NF):
                    sl = pl.ds(j * 16, 16)
                    v = arow[r, sl] + brow[r, sl]
                    arow[r, sl] = jnp.maximum(v * inv[i] + bch[j], 0.0)
            return 0
        lax.fori_loop(0, PWIN // 16, row_grp, 0)

        pltpu.sync_copy(arow, psh.at[bidx.at[0]], add=True)
        pltpu.sync_copy(ones_v, csh.at[bidx.at[0]], add=True)
        return 0
    lax.fori_loop(0, PNW, win_body, 0)
    plsc.subcore_barrier()

    @pl.when(s < 2)
    def _():
        pltpu.sync_copy(psh.at[pl.ds(s * (PG // 2), PG // 2), :],
                        psum_hbm.at[pl.ds(c * PG + s * (PG // 2), PG // 2), :])
        pltpu.sync_copy(csh.at[pl.ds(s * (PG // 2), PG // 2)],
                        pcnt_hbm.at[pl.ds(c * PG + s * (PG // 2), PG // 2)])


def _make_pool_kernel():
    mesh = plsc.VectorSubcoreMesh(core_axis_name="c", subcore_axis_name="s")
    return pl.kernel(
        _pool_body,
        out_type=(
            jax.ShapeDtypeStruct((2 * PG, H), jnp.float32),
            jax.ShapeDtypeStruct((2 * PG,), jnp.float32),
        ),
        mesh=mesh,
        compiler_params=pltpu.CompilerParams(use_tc_tiling_on_sc=False),
        scratch_types=[
            pltpu.VMEM((PWIN, H), jnp.float32),
            pltpu.VMEM((PWIN, H), jnp.float32),
            pltpu.VMEM((2 * PWIN,), jnp.float32),
            pltpu.VMEM((1, PWIN), jnp.int32),
            pltpu.VMEM((PWIN,), jnp.float32),
            pltpu.VMEM((H,), jnp.float32),
            pltpu.VMEM_SHARED((PG, H), jnp.float32),
            pltpu.VMEM_SHARED((PG,), jnp.float32),
        ],
    )


def _head_body(ps_ref, pc_ref, wp_ref, bp_ref, o_ref):
    sums = ps_ref[0, :G, :] + ps_ref[1, :G, :]
    cnts = pc_ref[0, 0, :G] + pc_ref[1, 0, :G]
    pooled = sums / jnp.maximum(cnts, 1.0)[:, None]
    o_ref[...] = jnp.dot(pooled, wp_ref[...],
                         preferred_element_type=jnp.float32) + bp_ref[...]


def _split_parts(accp, denp):
    parts = [accp[(c * NF + j) * N:(c * NF + j + 1) * N]
             for c in range(2) for j in range(NF)]
    den0 = jnp.pad(denp[:N], (0, NPAD - N),
                   constant_values=1.0).reshape(NB, 1, R)
    den1 = jnp.pad(denp[N:], (0, NPAD - N),
                   constant_values=1.0).reshape(NB, 1, R)
    return parts, (den0, den1)


def kernel(x, edge_index, batch, W1, a_src1, a_dst1, b1, W2, a_src2, a_dst2,
           b2, W3, a_src3, a_dst3, b3, Wp, bp):
    f32 = jnp.float32
    loop = jnp.arange(N, dtype=edge_index.dtype)
    pad_e = EP - (E + N)
    srcp = jnp.concatenate([edge_index[0], loop,
                            jnp.zeros((pad_e,), jnp.int32)])
    dstp = jnp.concatenate([edge_index[1], loop,
                            jnp.full((pad_e,), N, jnp.int32)])

    x_pad = jnp.pad(x, ((0, 0), (0, 5)))
    W1p = jnp.pad(W1, ((0, 5), (0, 0)))

    z16 = jnp.zeros((10000, 16), f32)
    z1 = jnp.zeros((10000,), f32)
    dummy_den = jnp.ones((NB, 1, R), f32)

    edge_k = _make_edge_kernel()

    def run_layer(first, parts, dens, b_prev, W, a_src, a_dst):
        h, as_, ad_, mx = _dense_layer(first, parts, dens, b_prev, W,
                                       a_src, a_dst)
        hcat = jnp.concatenate([h[:, j * 16:(j + 1) * 16]
                                for j in range(NF)], axis=0)
        ad_pad = jnp.pad(ad_, (0, 16))
        accp, denp, _ = edge_k(hcat, as_, ad_pad, mx, srcp, dstp, z16, z1)
        return _split_parts(accp, denp)

    parts1, dens1 = run_layer(True, [x_pad], (dummy_den, dummy_den),
                              b1, W1p, a_src1, a_dst1)
    parts2, dens2 = run_layer(False, parts1, dens1, b1, W2, a_src2, a_dst2)
    parts3, dens3 = run_layer(False, parts2, dens2, b2, W3, a_src3, a_dst3)

    a0 = jnp.concatenate(parts3[:NF], axis=0)
    a1 = jnp.concatenate(parts3[NF:], axis=0)
    a0 = jnp.pad(a0.reshape(NF, N, 16), ((0, 0), (0, NPAD - N), (0, 0))
                 ).reshape(NF * NPAD, 16)
    a1 = jnp.pad(a1.reshape(NF, N, 16), ((0, 0), (0, NPAD - N), (0, 0))
                 ).reshape(NF * NPAD, 16)
    dnp = jnp.concatenate([
        jnp.pad(dens3[0].reshape(NPAD)[:N], (0, NPAD - N), constant_values=1.0),
        jnp.pad(dens3[1].reshape(NPAD)[:N], (0, NPAD - N), constant_values=1.0),
    ])
    batchp = jnp.pad(batch, (0, NPAD - N), constant_values=G)
    zp = jnp.zeros((PG // 2, H), f32)
    zc = jnp.zeros((PG // 2,), f32)
    pool_k = _make_pool_kernel()
    psum, pcnt = pool_k(a0, a1, dnp, b3, batchp, zp, zc)

    Wpp = jnp.zeros((H, 128), f32).at[:, :NUM_OUT].set(Wp)
    bpp = jnp.zeros((1, 128), f32).at[0, :NUM_OUT].set(bp)
    out_pad = pl.pallas_call(
        _head_body,
        out_shape=jax.ShapeDtypeStruct((G, 128), f32),
    )(psum.reshape(2, PG, H), pcnt.reshape(2, 1, PG), Wpp, bpp)
    return out_pad[:, :NUM_OUT]


# speedup vs baseline: 22.5598x; 1.0204x over previous
import functools

import jax
import jax.numpy as jnp
from jax import lax
from jax.experimental import pallas as pl
from jax.experimental.pallas import tpu as pltpu
from jax.experimental.pallas import tpu_sc as plsc

N = 100000
E = 1600000
G = 256
H = 64
NUM_OUT = 5

K = 1024
ET = 53248
WN = ET // K
EP = 32 * ET
NF = H // 16
ND = N + 16

R = 2048
NB = 49
NPAD = NB * R

PW = 3136
PWIN = 112
PNW = PW // PWIN
PG = 272


def _iota16():
    return lax.iota(jnp.int32, 16)


def _dense_body(first, refs):
    i = pl.program_id(0)
    if first:
        (x_ref, d0_ref, d1_ref, bp_ref, w_ref, asrc_ref, adst_ref,
         h_ref, as_ref, ad_ref, mx_ref) = refs
        act = x_ref[...]
    else:
        (a00, a01, a02, a03, a10, a11, a12, a13, d0_ref, d1_ref, bp_ref,
         w_ref, asrc_ref, adst_ref, h_ref, as_ref, ad_ref, mx_ref) = refs
        num = jnp.concatenate(
            [a00[...] + a10[...], a01[...] + a11[...],
             a02[...] + a12[...], a03[...] + a13[...]], axis=1)
        den = (d0_ref[...] + d1_ref[...]).reshape(R, 1)
        act = jnp.maximum(num / den + bp_ref[...], 0.0)
    h = jnp.dot(act, w_ref[...], preferred_element_type=jnp.float32)
    h_ref[...] = h
    asb = jnp.dot(h, asrc_ref[...], preferred_element_type=jnp.float32)[:, 0]
    adb = jnp.dot(h, adst_ref[...], preferred_element_type=jnp.float32)[:, 0]
    as_ref[...] = asb.reshape(1, 1, R)
    ad_ref[...] = adb.reshape(1, 1, R)
    valid = i * R + lax.iota(jnp.int32, R) < N
    bmax = jnp.max(jnp.where(valid, asb, -3.4e38))

    @pl.when(i == 0)
    def _():
        mx_ref[...] = jnp.full((1, 1), -3.4e38, jnp.float32)
    mx_ref[...] = jnp.maximum(mx_ref[...], bmax)


def _dense_layer(first, acc_parts, den_pair, b_prev, W, a_src, a_dst):
    kin = acc_parts[0].shape[1] if first else H
    grid = (NB,)
    out_shapes = (
        jax.ShapeDtypeStruct((N, H), jnp.float32),
        jax.ShapeDtypeStruct((NB, 1, R), jnp.float32),
        jax.ShapeDtypeStruct((NB, 1, R), jnp.float32),
        jax.ShapeDtypeStruct((1, 1), jnp.float32),
    )
    part_specs = ([pl.BlockSpec((R, kin), lambda i: (i, 0))] if first else
                  [pl.BlockSpec((R, 16), lambda i: (i, 0))] * 8)
    in_specs = part_specs + [
        pl.BlockSpec((1, 1, R), lambda i: (i, 0, 0)),
        pl.BlockSpec((1, 1, R), lambda i: (i, 0, 0)),
        pl.BlockSpec((1, H), lambda i: (0, 0)),
        pl.BlockSpec((kin, H), lambda i: (0, 0)),
        pl.BlockSpec((H, 1), lambda i: (0, 0)),
        pl.BlockSpec((H, 1), lambda i: (0, 0)),
    ]
    out_specs = (
        pl.BlockSpec((R, H), lambda i: (i, 0)),
        pl.BlockSpec((1, 1, R), lambda i: (i, 0, 0)),
        pl.BlockSpec((1, 1, R), lambda i: (i, 0, 0)),
        pl.BlockSpec((1, 1), lambda i: (0, 0)),
    )

    def body(*refs):
        _dense_body(first, refs)

    h, as3d, ad3d, mx = pl.pallas_call(
        body,
        grid=grid,
        in_specs=in_specs,
        out_specs=out_specs,
        out_shape=out_shapes,
    )(*acc_parts, den_pair[0], den_pair[1], b_prev.reshape(1, H), W,
      a_src.reshape(H, 1), a_dst.reshape(H, 1))
    as_flat = as3d.reshape(NPAD)[:N]
    ad_flat = ad3d.reshape(NPAD)[:N]
    mx16 = jnp.broadcast_to(mx.reshape(1), (16,))
    return h, as_flat, ad_flat, mx16


def _edge_body(hcat_hbm, as_hbm, ad_hbm, mx_hbm, src_hbm, dst_hbm,
               z16_hbm, z1_hbm,
               acc_hbm, den_hbm, w_hbm,
               srcw, dstw, asv, adv, wbuf, gix, dli, hsl, mxv, acc_sh,
               den_sh, sem, sem2, sem3):
    c = lax.axis_index("c")
    s = lax.axis_index("s")
    w32 = c * 16 + s
    base_e = w32 * ET
    pltpu.sync_copy(mx_hbm, mxv)
    asmax = mxv[pl.ds(0, 16)][0]

    def pass_body(p, _):
        @pl.when(s < 10)
        def _():
            pltpu.sync_copy(z16_hbm, acc_sh.at[pl.ds(s * 10000, 10000), :])

        @pl.when(s == 10)
        def _():
            pltpu.sync_copy(z16_hbm.at[pl.ds(0, 16), :],
                            acc_sh.at[pl.ds(N, 16), :])

        @pl.when(p == 0)
        def _():
            @pl.when(s < 10)
            def _():
                pltpu.sync_copy(z1_hbm, den_sh.at[pl.ds(s * 10000, 10000)])

            @pl.when(s == 10)
            def _():
                pltpu.sync_copy(z1_hbm.at[pl.ds(0, 16)],
                                den_sh.at[pl.ds(N, 16)])
        plsc.subcore_barrier()

        def window_body(wi, _):
            eb = base_e + wi * K
            d1 = pltpu.async_copy(src_hbm.at[pl.ds(eb, K)], srcw, sem)
            d2 = pltpu.async_copy(dst_hbm.at[pl.ds(eb, K)], dstw, sem)
            d1.wait()
            d2.wait()
            for t in range(K // 16):
                sl = pl.ds((t % 8) * 16, 16)
                gix[t // 8, sl] = srcw[pl.ds(t * 16, 16)] + p * N
                dli[t // 8, sl] = dstw[pl.ds(t * 16, 16)]
            descs = []
            for j in range(K // 128):
                descs.append(pltpu.async_copy(
                    hcat_hbm.at[gix.at[j]],
                    hsl.at[pl.ds(j * 128, 128), :], sem))

            @pl.when(p == 0)
            def _():
                adescs = []
                for j in range(K // 128):
                    adescs.append(pltpu.async_copy(
                        as_hbm.at[srcw.at[pl.ds(j * 128, 128)]],
                        asv.at[pl.ds(j * 128, 128)], sem3))
                    adescs.append(pltpu.async_copy(
                        ad_hbm.at[dstw.at[pl.ds(j * 128, 128)]],
                        adv.at[pl.ds(j * 128, 128)], sem3))
                for d in adescs:
                    d.wait()
                for t in range(K // 16):
                    sl = pl.ds(t * 16, 16)
                    av = asv[sl]
                    bv = adv[sl]
                    z = av + bv
                    e = jnp.maximum(z, 0.2 * z)
                    u = asmax + bv
                    cb = jnp.maximum(u, 0.2 * u)
                    wbuf[sl] = jnp.exp(e - cb)
                pltpu.async_copy(wbuf, w_hbm.at[pl.ds(eb, K)], sem2).wait()

            @pl.when(p != 0)
            def _():
                pltpu.async_copy(w_hbm.at[pl.ds(eb, K)], wbuf, sem2).wait()
            for d in descs:
                d.wait()

            def scale_grp(gb, _):
                wv = wbuf[pl.ds(gb * 16, 16)]
                for i in range(16):
                    r = gb * 16 + i
                    hsl[r, pl.ds(0, 16)] = hsl[r, pl.ds(0, 16)] * wv[i]
                return 0
            lax.fori_loop(0, K // 16, scale_grp, 0)

            sdescs = []
            for j in range(K // 128):
                sdescs.append(pltpu.async_copy(
                    hsl.at[pl.ds(j * 128, 128), :],
                    acc_sh.at[dli.at[j]], sem2, add=True))

            @pl.when(p == 0)
            def _():
                ddescs = []
                for j in range(K // 128):
                    ddescs.append(pltpu.async_copy(
                        wbuf.at[pl.ds(j * 128, 128)],
                        den_sh.at[dli.at[j]], sem2, add=True))
                for d in ddescs:
                    d.wait()
            for d in sdescs:
                d.wait()
            return 0

        lax.fori_loop(0, WN, window_body, 0)
        plsc.subcore_barrier()

        @pl.when(s < 10)
        def _():
            pltpu.sync_copy(
                acc_sh.at[pl.ds(s * 10000, 10000), :],
                acc_hbm.at[pl.ds((c * NF + p) * N + s * 10000, 10000), :])

            @pl.when(p == 0)
            def _():
                pltpu.sync_copy(
                    den_sh.at[pl.ds(s * 10000, 10000)],
                    den_hbm.at[pl.ds(c * N + s * 10000, 10000)])
        plsc.subcore_barrier()
        return 0

    lax.fori_loop(0, NF, pass_body, 0)


def _make_edge_kernel():
    mesh = plsc.VectorSubcoreMesh(core_axis_name="c", subcore_axis_name="s")
    return pl.kernel(
        _edge_body,
        out_type=(
            jax.ShapeDtypeStruct((2 * NF * N, 16), jnp.float32),
            jax.ShapeDtypeStruct((2 * N,), jnp.float32),
            jax.ShapeDtypeStruct((EP,), jnp.float32),
        ),
        mesh=mesh,
        compiler_params=pltpu.CompilerParams(use_tc_tiling_on_sc=False),
        scratch_types=[
            pltpu.VMEM((K,), jnp.int32),
            pltpu.VMEM((K,), jnp.int32),
            pltpu.VMEM((K,), jnp.float32),
            pltpu.VMEM((K,), jnp.float32),
            pltpu.VMEM((K,), jnp.float32),
            pltpu.VMEM((K // 128, 128), jnp.int32),
            pltpu.VMEM((K // 128, 128), jnp.int32),
            pltpu.VMEM((K, 16), jnp.float32),
            pltpu.VMEM((16,), jnp.float32),
            pltpu.VMEM_SHARED((ND, 16), jnp.float32),
            pltpu.VMEM_SHARED((ND,), jnp.float32),
            pltpu.SemaphoreType.DMA,
            pltpu.SemaphoreType.DMA,
            pltpu.SemaphoreType.DMA,
        ],
    )


def _pool_body(a0_hbm, a1_hbm, d_hbm, b_hbm, batch_hbm, zp_hbm, zc_hbm,
               psum_hbm, pcnt_hbm,
               arow, brow, drow, bidx, ones_v, bvec, psh, csh):
    c = lax.axis_index("c")
    s = lax.axis_index("s")
    w = s * 2 + c
    pltpu.sync_copy(b_hbm, bvec)
    bch = [bvec[pl.ds(j * 16, 16)] for j in range(NF)]
    for t in range(PWIN // 16):
        ones_v[pl.ds(t * 16, 16)] = jnp.ones((16,), jnp.float32)

    @pl.when(s < 2)
    def _():
        pltpu.sync_copy(zp_hbm, psh.at[pl.ds(s * (PG // 2), PG // 2), :])
        pltpu.sync_copy(zc_hbm, csh.at[pl.ds(s * (PG // 2), PG // 2)])
    plsc.subcore_barrier()

    def win_body(wi, _):
        nb = w * PW + wi * PWIN
        for j in range(NF):
            pltpu.sync_copy(a0_hbm.at[pl.ds(j * NPAD + nb, PWIN), :],
                            arow.at[pl.ds(0, PWIN), pl.ds(j * 16, 16)])
            pltpu.sync_copy(a1_hbm.at[pl.ds(j * NPAD + nb, PWIN), :],
                            brow.at[pl.ds(0, PWIN), pl.ds(j * 16, 16)])
        pltpu.sync_copy(d_hbm.at[pl.ds(nb, PWIN)], drow.at[pl.ds(0, PWIN)])
        pltpu.sync_copy(d_hbm.at[pl.ds(NPAD + nb, PWIN)],
                        drow.at[pl.ds(PWIN, PWIN)])
        pltpu.sync_copy(batch_hbm.at[pl.ds(nb, PWIN)], bidx.at[0])

        def row_grp(t, _):
            dv = (drow[pl.ds(t * 16, 16)] +
                  drow[pl.ds(PWIN + t * 16, 16)])
            inv = 1.0 / dv
            for i in range(16):
                r = t * 16 + i
                for j in range(NF):
                    sl = pl.ds(j * 16, 16)
                    v = arow[r, sl] + brow[r, sl]
                    arow[r, sl] = jnp.maximum(v * inv[i] + bch[j], 0.0)
            return 0
        lax.fori_loop(0, PWIN // 16, row_grp, 0)

        pltpu.sync_copy(arow, psh.at[bidx.at[0]], add=True)
        pltpu.sync_copy(ones_v, csh.at[bidx.at[0]], add=True)
        return 0
    lax.fori_loop(0, PNW, win_body, 0)
    plsc.subcore_barrier()

    @pl.when(s < 2)
    def _():
        pltpu.sync_copy(psh.at[pl.ds(s * (PG // 2), PG // 2), :],
                        psum_hbm.at[pl.ds(c * PG + s * (PG // 2), PG // 2), :])
        pltpu.sync_copy(csh.at[pl.ds(s * (PG // 2), PG // 2)],
                        pcnt_hbm.at[pl.ds(c * PG + s * (PG // 2), PG // 2)])


def _make_pool_kernel():
    mesh = plsc.VectorSubcoreMesh(core_axis_name="c", subcore_axis_name="s")
    return pl.kernel(
        _pool_body,
        out_type=(
            jax.ShapeDtypeStruct((2 * PG, H), jnp.float32),
            jax.ShapeDtypeStruct((2 * PG,), jnp.float32),
        ),
        mesh=mesh,
        compiler_params=pltpu.CompilerParams(use_tc_tiling_on_sc=False),
        scratch_types=[
            pltpu.VMEM((PWIN, H), jnp.float32),
            pltpu.VMEM((PWIN, H), jnp.float32),
            pltpu.VMEM((2 * PWIN,), jnp.float32),
            pltpu.VMEM((1, PWIN), jnp.int32),
            pltpu.VMEM((PWIN,), jnp.float32),
            pltpu.VMEM((H,), jnp.float32),
            pltpu.VMEM_SHARED((PG, H), jnp.float32),
            pltpu.VMEM_SHARED((PG,), jnp.float32),
        ],
    )


def _head_body(ps_ref, pc_ref, wp_ref, bp_ref, o_ref):
    sums = ps_ref[0, :G, :] + ps_ref[1, :G, :]
    cnts = pc_ref[0, 0, :G] + pc_ref[1, 0, :G]
    pooled = sums / jnp.maximum(cnts, 1.0)[:, None]
    o_ref[...] = jnp.dot(pooled, wp_ref[...],
                         preferred_element_type=jnp.float32) + bp_ref[...]


def _split_parts(accp, denp):
    parts = [accp[(c * NF + j) * N:(c * NF + j + 1) * N]
             for c in range(2) for j in range(NF)]
    den0 = jnp.pad(denp[:N], (0, NPAD - N),
                   constant_values=1.0).reshape(NB, 1, R)
    den1 = jnp.pad(denp[N:], (0, NPAD - N),
                   constant_values=1.0).reshape(NB, 1, R)
    return parts, (den0, den1)


def kernel(x, edge_index, batch, W1, a_src1, a_dst1, b1, W2, a_src2, a_dst2,
           b2, W3, a_src3, a_dst3, b3, Wp, bp):
    f32 = jnp.float32
    loop = jnp.arange(N, dtype=edge_index.dtype)
    pad_e = EP - (E + N)
    srcp = jnp.concatenate([edge_index[0], loop,
                            jnp.zeros((pad_e,), jnp.int32)])
    dstp = jnp.concatenate([edge_index[1], loop,
                            jnp.full((pad_e,), N, jnp.int32)])

    x_pad = jnp.pad(x, ((0, 0), (0, 5)))
    W1p = jnp.pad(W1, ((0, 5), (0, 0)))

    z16 = jnp.zeros((10000, 16), f32)
    z1 = jnp.zeros((10000,), f32)
    dummy_den = jnp.ones((NB, 1, R), f32)

    edge_k = _make_edge_kernel()

    def run_layer(first, parts, dens, b_prev, W, a_src, a_dst):
        h, as_, ad_, mx = _dense_layer(first, parts, dens, b_prev, W,
                                       a_src, a_dst)
        hcat = jnp.concatenate([h[:, j * 16:(j + 1) * 16]
                                for j in range(NF)], axis=0)
        ad_pad = jnp.pad(ad_, (0, 16))
        accp, denp, _ = edge_k(hcat, as_, ad_pad, mx, srcp, dstp, z16, z1)
        return _split_parts(accp, denp)

    parts1, dens1 = run_layer(True, [x_pad], (dummy_den, dummy_den),
                              b1, W1p, a_src1, a_dst1)
    parts2, dens2 = run_layer(False, parts1, dens1, b1, W2, a_src2, a_dst2)
    parts3, dens3 = run_layer(False, parts2, dens2, b2, W3, a_src3, a_dst3)

    a0 = jnp.concatenate(parts3[:NF], axis=0)
    a1 = jnp.concatenate(parts3[NF:], axis=0)
    a0 = jnp.pad(a0.reshape(NF, N, 16), ((0, 0), (0, NPAD - N), (0, 0))
                 ).reshape(NF * NPAD, 16)
    a1 = jnp.pad(a1.reshape(NF, N, 16), ((0, 0), (0, NPAD - N), (0, 0))
                 ).reshape(NF * NPAD, 16)
    dnp = jnp.concatenate([
        jnp.pad(dens3[0].reshape(NPAD)[:N], (0, NPAD - N), constant_values=1.0),
        jnp.pad(dens3[1].reshape(NPAD)[:N], (0, NPAD - N), constant_values=1.0),
    ])
    batchp = jnp.pad(batch, (0, NPAD - N), constant_values=G)
    zp = jnp.zeros((PG // 2, H), f32)
    zc = jnp.zeros((PG // 2,), f32)
    pool_k = _make_pool_kernel()
    psum, pcnt = pool_k(a0, a1, dnp, b3, batchp, zp, zc)

    Wpp = jnp.zeros((H, 128), f32).at[:, :NUM_OUT].set(Wp)
    bpp = jnp.zeros((1, 128), f32).at[0, :NUM_OUT].set(bp)
    out_pad = pl.pallas_call(
        _head_body,
        out_shape=jax.ShapeDtypeStruct((G, 128), f32),
    )(psum.reshape(2, PG, H), pcnt.reshape(2, 1, PG), Wpp, bpp)
    return out_pad[:, :NUM_OUT]


# speedup vs baseline: 22.6537x; 1.0042x over previous
import functools

import jax
import jax.numpy as jnp
from jax import lax
from jax.experimental import pallas as pl
from jax.experimental.pallas import tpu as pltpu
from jax.experimental.pallas import tpu_sc as plsc

N = 100000
E = 1600000
G = 256
H = 64
NUM_OUT = 5

K = 512
ET = 53248
WN = ET // K
EP = 32 * ET
NF = H // 16
ND = N + 16

R = 2048
NB = 49
NPAD = NB * R

PW = 3136
PWIN = 112
PNW = PW // PWIN
PG = 272


def _iota16():
    return lax.iota(jnp.int32, 16)


def _dense_body(first, refs):
    i = pl.program_id(0)
    if first:
        (x_ref, d0_ref, d1_ref, bp_ref, w_ref, asrc_ref, adst_ref,
         h_ref, as_ref, ad_ref, mx_ref) = refs
        act = x_ref[...]
    else:
        (a00, a01, a02, a03, a10, a11, a12, a13, d0_ref, d1_ref, bp_ref,
         w_ref, asrc_ref, adst_ref, h_ref, as_ref, ad_ref, mx_ref) = refs
        num = jnp.concatenate(
            [a00[...] + a10[...], a01[...] + a11[...],
             a02[...] + a12[...], a03[...] + a13[...]], axis=1)
        den = (d0_ref[...] + d1_ref[...]).reshape(R, 1)
        act = jnp.maximum(num / den + bp_ref[...], 0.0)
    h = jnp.dot(act, w_ref[...], preferred_element_type=jnp.float32)
    h_ref[...] = h
    asb = jnp.dot(h, asrc_ref[...], preferred_element_type=jnp.float32)[:, 0]
    adb = jnp.dot(h, adst_ref[...], preferred_element_type=jnp.float32)[:, 0]
    as_ref[...] = asb.reshape(1, 1, R)
    ad_ref[...] = adb.reshape(1, 1, R)
    valid = i * R + lax.iota(jnp.int32, R) < N
    bmax = jnp.max(jnp.where(valid, asb, -3.4e38))

    @pl.when(i == 0)
    def _():
        mx_ref[...] = jnp.full((1, 1), -3.4e38, jnp.float32)
    mx_ref[...] = jnp.maximum(mx_ref[...], bmax)


def _dense_layer(first, acc_parts, den_pair, b_prev, W, a_src, a_dst):
    kin = acc_parts[0].shape[1] if first else H
    grid = (NB,)
    out_shapes = (
        jax.ShapeDtypeStruct((N, H), jnp.float32),
        jax.ShapeDtypeStruct((NB, 1, R), jnp.float32),
        jax.ShapeDtypeStruct((NB, 1, R), jnp.float32),
        jax.ShapeDtypeStruct((1, 1), jnp.float32),
    )
    part_specs = ([pl.BlockSpec((R, kin), lambda i: (i, 0))] if first else
                  [pl.BlockSpec((R, 16), lambda i: (i, 0))] * 8)
    in_specs = part_specs + [
        pl.BlockSpec((1, 1, R), lambda i: (i, 0, 0)),
        pl.BlockSpec((1, 1, R), lambda i: (i, 0, 0)),
        pl.BlockSpec((1, H), lambda i: (0, 0)),
        pl.BlockSpec((kin, H), lambda i: (0, 0)),
        pl.BlockSpec((H, 1), lambda i: (0, 0)),
        pl.BlockSpec((H, 1), lambda i: (0, 0)),
    ]
    out_specs = (
        pl.BlockSpec((R, H), lambda i: (i, 0)),
        pl.BlockSpec((1, 1, R), lambda i: (i, 0, 0)),
        pl.BlockSpec((1, 1, R), lambda i: (i, 0, 0)),
        pl.BlockSpec((1, 1), lambda i: (0, 0)),
    )

    def body(*refs):
        _dense_body(first, refs)

    h, as3d, ad3d, mx = pl.pallas_call(
        body,
        grid=grid,
        in_specs=in_specs,
        out_specs=out_specs,
        out_shape=out_shapes,
    )(*acc_parts, den_pair[0], den_pair[1], b_prev.reshape(1, H), W,
      a_src.reshape(H, 1), a_dst.reshape(H, 1))
    as_flat = as3d.reshape(NPAD)[:N]
    ad_flat = ad3d.reshape(NPAD)[:N]
    mx16 = jnp.broadcast_to(mx.reshape(1), (16,))
    return h, as_flat, ad_flat, mx16


def _edge_body(hcat_hbm, as_hbm, ad_hbm, mx_hbm, src_hbm, dst_hbm,
               z16_hbm, z1_hbm,
               acc_hbm, den_hbm, w_hbm,
               srcw, dstw, asv, adv,
               wbuf0, gix0, dli0, hsl0,
               wbuf1, gix1, dli1, hsl1,
               mxv, acc_sh, den_sh,
               semL, sem3, semw, semg0, semg1, sems0, sems1):
    bufs = [(wbuf0, gix0, dli0, hsl0), (wbuf1, gix1, dli1, hsl1)]
    semg = [semg0, semg1]
    sems = [sems0, sems1]
    c = lax.axis_index("c")
    s = lax.axis_index("s")
    w32 = c * 16 + s
    base_e = w32 * ET
    pltpu.sync_copy(mx_hbm, mxv)
    asmax = mxv[pl.ds(0, 16)][0]

    def pass_body(p, _):
        @pl.when(s < 10)
        def _():
            pltpu.sync_copy(z16_hbm, acc_sh.at[pl.ds(s * 10000, 10000), :])

        @pl.when(s == 10)
        def _():
            pltpu.sync_copy(z16_hbm.at[pl.ds(0, 16), :],
                            acc_sh.at[pl.ds(N, 16), :])

        @pl.when(p == 0)
        def _():
            @pl.when(s < 10)
            def _():
                pltpu.sync_copy(z1_hbm, den_sh.at[pl.ds(s * 10000, 10000)])

            @pl.when(s == 10)
            def _():
                pltpu.sync_copy(z1_hbm.at[pl.ds(0, 16)],
                                den_sh.at[pl.ds(N, 16)])
        plsc.subcore_barrier()

        def stage1(eb, bb):
            wbuf, gix, dli, hsl = bufs[bb]
            d1 = pltpu.async_copy(src_hbm.at[pl.ds(eb, K)], srcw, semL)
            d2 = pltpu.async_copy(dst_hbm.at[pl.ds(eb, K)], dstw, semL)
            d1.wait()
            d2.wait()
            for t in range(K // 16):
                sl = pl.ds((t % 8) * 16, 16)
                gix[t // 8, sl] = srcw[pl.ds(t * 16, 16)] + p * N
                dli[t // 8, sl] = dstw[pl.ds(t * 16, 16)]
            hdescs = []
            for j in range(K // 128):
                hdescs.append(pltpu.async_copy(
                    hcat_hbm.at[gix.at[j]],
                    hsl.at[pl.ds(j * 128, 128), :], semg[bb]))

            @pl.when(p == 0)
            def _():
                adescs = []
                for j in range(K // 128):
                    adescs.append(pltpu.async_copy(
                        as_hbm.at[srcw.at[pl.ds(j * 128, 128)]],
                        asv.at[pl.ds(j * 128, 128)], sem3))
                    adescs.append(pltpu.async_copy(
                        ad_hbm.at[dstw.at[pl.ds(j * 128, 128)]],
                        adv.at[pl.ds(j * 128, 128)], sem3))
                for d in adescs:
                    d.wait()
                for t in range(K // 16):
                    sl = pl.ds(t * 16, 16)
                    av = asv[sl]
                    bv = adv[sl]
                    z = av + bv
                    e = jnp.maximum(z, 0.2 * z)
                    u = asmax + bv
                    cb = jnp.maximum(u, 0.2 * u)
                    wbuf[sl] = jnp.exp(e - cb)
                pltpu.async_copy(wbuf, w_hbm.at[pl.ds(eb, K)], semw).wait()

            @pl.when(p != 0)
            def _():
                pltpu.async_copy(w_hbm.at[pl.ds(eb, K)], wbuf, semw).wait()
            return hdescs

        def stage2(bb, hdescs):
            wbuf, gix, dli, hsl = bufs[bb]
            for d in hdescs:
                d.wait()

            def scale_grp(gb, _):
                wv = wbuf[pl.ds(gb * 16, 16)]
                for i in range(16):
                    r = gb * 16 + i
                    hsl[r, pl.ds(0, 16)] = hsl[r, pl.ds(0, 16)] * wv[i]
                return 0
            lax.fori_loop(0, K // 16, scale_grp, 0)

            sdescs = []
            for j in range(K // 128):
                sdescs.append(pltpu.async_copy(
                    hsl.at[pl.ds(j * 128, 128), :],
                    acc_sh.at[dli.at[j]], sems[bb], add=True))

            @pl.when(p == 0)
            def _():
                ddescs = []
                for j in range(K // 128):
                    ddescs.append(pltpu.async_copy(
                        wbuf.at[pl.ds(j * 128, 128)],
                        den_sh.at[dli.at[j]], sems[bb], add=True))
                for d in ddescs:
                    d.wait()
            return sdescs

        def pair_body(pi, _):
            eb = base_e + pi * (2 * K)
            hA = stage1(eb, 0)
            hB = stage1(eb + K, 1)
            sA = stage2(0, hA)
            sB = stage2(1, hB)
            for d in sA:
                d.wait()
            for d in sB:
                d.wait()
            return 0

        lax.fori_loop(0, WN // 2, pair_body, 0)
        plsc.subcore_barrier()

        @pl.when(s < 10)
        def _():
            pltpu.sync_copy(
                acc_sh.at[pl.ds(s * 10000, 10000), :],
                acc_hbm.at[pl.ds((c * NF + p) * N + s * 10000, 10000), :])

            @pl.when(p == 0)
            def _():
                pltpu.sync_copy(
                    den_sh.at[pl.ds(s * 10000, 10000)],
                    den_hbm.at[pl.ds(c * N + s * 10000, 10000)])
        plsc.subcore_barrier()
        return 0

    lax.fori_loop(0, NF, pass_body, 0)


def _make_edge_kernel():
    mesh = plsc.VectorSubcoreMesh(core_axis_name="c", subcore_axis_name="s")
    return pl.kernel(
        _edge_body,
        out_type=(
            jax.ShapeDtypeStruct((2 * NF * N, 16), jnp.float32),
            jax.ShapeDtypeStruct((2 * N,), jnp.float32),
            jax.ShapeDtypeStruct((EP,), jnp.float32),
        ),
        mesh=mesh,
        compiler_params=pltpu.CompilerParams(use_tc_tiling_on_sc=False),
        scratch_types=(
            [pltpu.VMEM((K,), jnp.int32),
             pltpu.VMEM((K,), jnp.int32),
             pltpu.VMEM((K,), jnp.float32),
             pltpu.VMEM((K,), jnp.float32)] +
            [pltpu.VMEM((K,), jnp.float32),
             pltpu.VMEM((K // 128, 128), jnp.int32),
             pltpu.VMEM((K // 128, 128), jnp.int32),
             pltpu.VMEM((K, 16), jnp.float32)] * 2 +
            [pltpu.VMEM((16,), jnp.float32),
             pltpu.VMEM_SHARED((ND, 16), jnp.float32),
             pltpu.VMEM_SHARED((ND,), jnp.float32)] +
            [pltpu.SemaphoreType.DMA] * 7
        ),
    )


def _pool_body(a0_hbm, a1_hbm, d_hbm, b_hbm, batch_hbm, zp_hbm, zc_hbm,
               psum_hbm, pcnt_hbm,
               arow, brow, drow, bidx, ones_v, bvec, psh, csh):
    c = lax.axis_index("c")
    s = lax.axis_index("s")
    w = s * 2 + c
    pltpu.sync_copy(b_hbm, bvec)
    bch = [bvec[pl.ds(j * 16, 16)] for j in range(NF)]
    for t in range(PWIN // 16):
        ones_v[pl.ds(t * 16, 16)] = jnp.ones((16,), jnp.float32)

    @pl.when(s < 2)
    def _():
        pltpu.sync_copy(zp_hbm, psh.at[pl.ds(s * (PG // 2), PG // 2), :])
        pltpu.sync_copy(zc_hbm, csh.at[pl.ds(s * (PG // 2), PG // 2)])
    plsc.subcore_barrier()

    def win_body(wi, _):
        nb = w * PW + wi * PWIN
        for j in range(NF):
            pltpu.sync_copy(a0_hbm.at[pl.ds(j * NPAD + nb, PWIN), :],
                            arow.at[pl.ds(0, PWIN), pl.ds(j * 16, 16)])
            pltpu.sync_copy(a1_hbm.at[pl.ds(j * NPAD + nb, PWIN), :],
                            brow.at[pl.ds(0, PWIN), pl.ds(j * 16, 16)])
        pltpu.sync_copy(d_hbm.at[pl.ds(nb, PWIN)], drow.at[pl.ds(0, PWIN)])
        pltpu.sync_copy(d_hbm.at[pl.ds(NPAD + nb, PWIN)],
                        drow.at[pl.ds(PWIN, PWIN)])
        pltpu.sync_copy(batch_hbm.at[pl.ds(nb, PWIN)], bidx.at[0])

        def row_grp(t, _):
            dv = (drow[pl.ds(t * 16, 16)] +
                  drow[pl.ds(PWIN + t * 16, 16)])
            inv = 1.0 / dv
            for i in range(16):
                r = t * 16 + i
                for j in range(NF):
                    sl = pl.ds(j * 16, 16)
                    v = arow[r, sl] + brow[r, sl]
                    arow[r, sl] = jnp.maximum(v * inv[i] + bch[j], 0.0)
            return 0
        lax.fori_loop(0, PWIN // 16, row_grp, 0)

        pltpu.sync_copy(arow, psh.at[bidx.at[0]], add=True)
        pltpu.sync_copy(ones_v, csh.at[bidx.at[0]], add=True)
        return 0
    lax.fori_loop(0, PNW, win_body, 0)
    plsc.subcore_barrier()

    @pl.when(s < 2)
    def _():
        pltpu.sync_copy(psh.at[pl.ds(s * (PG // 2), PG // 2), :],
                        psum_hbm.at[pl.ds(c * PG + s * (PG // 2), PG // 2), :])
        pltpu.sync_copy(csh.at[pl.ds(s * (PG // 2), PG // 2)],
                        pcnt_hbm.at[pl.ds(c * PG + s * (PG // 2), PG // 2)])


def _make_pool_kernel():
    mesh = plsc.VectorSubcoreMesh(core_axis_name="c", subcore_axis_name="s")
    return pl.kernel(
        _pool_body,
        out_type=(
            jax.ShapeDtypeStruct((2 * PG, H), jnp.float32),
            jax.ShapeDtypeStruct((2 * PG,), jnp.float32),
        ),
        mesh=mesh,
        compiler_params=pltpu.CompilerParams(use_tc_tiling_on_sc=False),
        scratch_types=[
            pltpu.VMEM((PWIN, H), jnp.float32),
            pltpu.VMEM((PWIN, H), jnp.float32),
            pltpu.VMEM((2 * PWIN,), jnp.float32),
            pltpu.VMEM((1, PWIN), jnp.int32),
            pltpu.VMEM((PWIN,), jnp.float32),
            pltpu.VMEM((H,), jnp.float32),
            pltpu.VMEM_SHARED((PG, H), jnp.float32),
            pltpu.VMEM_SHARED((PG,), jnp.float32),
        ],
    )


def _head_body(ps_ref, pc_ref, wp_ref, bp_ref, o_ref):
    sums = ps_ref[0, :G, :] + ps_ref[1, :G, :]
    cnts = pc_ref[0, 0, :G] + pc_ref[1, 0, :G]
    pooled = sums / jnp.maximum(cnts, 1.0)[:, None]
    o_ref[...] = jnp.dot(pooled, wp_ref[...],
                         preferred_element_type=jnp.float32) + bp_ref[...]


def _split_parts(accp, denp):
    parts = [accp[(c * NF + j) * N:(c * NF + j + 1) * N]
             for c in range(2) for j in range(NF)]
    den0 = jnp.pad(denp[:N], (0, NPAD - N),
                   constant_values=1.0).reshape(NB, 1, R)
    den1 = jnp.pad(denp[N:], (0, NPAD - N),
                   constant_values=1.0).reshape(NB, 1, R)
    return parts, (den0, den1)


def kernel(x, edge_index, batch, W1, a_src1, a_dst1, b1, W2, a_src2, a_dst2,
           b2, W3, a_src3, a_dst3, b3, Wp, bp):
    f32 = jnp.float32
    loop = jnp.arange(N, dtype=edge_index.dtype)
    pad_e = EP - (E + N)
    srcp = jnp.concatenate([edge_index[0], loop,
                            jnp.zeros((pad_e,), jnp.int32)])
    dstp = jnp.concatenate([edge_index[1], loop,
                            jnp.full((pad_e,), N, jnp.int32)])

    x_pad = jnp.pad(x, ((0, 0), (0, 5)))
    W1p = jnp.pad(W1, ((0, 5), (0, 0)))

    z16 = jnp.zeros((10000, 16), f32)
    z1 = jnp.zeros((10000,), f32)
    dummy_den = jnp.ones((NB, 1, R), f32)

    edge_k = _make_edge_kernel()

    def run_layer(first, parts, dens, b_prev, W, a_src, a_dst):
        h, as_, ad_, mx = _dense_layer(first, parts, dens, b_prev, W,
                                       a_src, a_dst)
        hcat = jnp.concatenate([h[:, j * 16:(j + 1) * 16]
                                for j in range(NF)], axis=0)
        ad_pad = jnp.pad(ad_, (0, 16))
        accp, denp, _ = edge_k(hcat, as_, ad_pad, mx, srcp, dstp, z16, z1)
        return _split_parts(accp, denp)

    parts1, dens1 = run_layer(True, [x_pad], (dummy_den, dummy_den),
                              b1, W1p, a_src1, a_dst1)
    parts2, dens2 = run_layer(False, parts1, dens1, b1, W2, a_src2, a_dst2)
    parts3, dens3 = run_layer(False, parts2, dens2, b2, W3, a_src3, a_dst3)

    a0 = jnp.concatenate(parts3[:NF], axis=0)
    a1 = jnp.concatenate(parts3[NF:], axis=0)
    a0 = jnp.pad(a0.reshape(NF, N, 16), ((0, 0), (0, NPAD - N), (0, 0))
                 ).reshape(NF * NPAD, 16)
    a1 = jnp.pad(a1.reshape(NF, N, 16), ((0, 0), (0, NPAD - N), (0, 0))
                 ).reshape(NF * NPAD, 16)
    dnp = jnp.concatenate([
        jnp.pad(dens3[0].reshape(NPAD)[:N], (0, NPAD - N), constant_values=1.0),
        jnp.pad(dens3[1].reshape(NPAD)[:N], (0, NPAD - N), constant_values=1.0),
    ])
    batchp = jnp.pad(batch, (0, NPAD - N), constant_values=G)
    zp = jnp.zeros((PG // 2, H), f32)
    zc = jnp.zeros((PG // 2,), f32)
    pool_k = _make_pool_kernel()
    psum, pcnt = pool_k(a0, a1, dnp, b3, batchp, zp, zc)

    Wpp = jnp.zeros((H, 128), f32).at[:, :NUM_OUT].set(Wp)
    bpp = jnp.zeros((1, 128), f32).at[0, :NUM_OUT].set(bp)
    out_pad = pl.pallas_call(
        _head_body,
        out_shape=jax.ShapeDtypeStruct((G, 128), f32),
    )(psum.reshape(2, PG, H), pcnt.reshape(2, 1, PG), Wpp, bpp)
    return out_pad[:, :NUM_OUT]


# speedup vs baseline: 23.0579x; 1.0178x over previous
import functools

import jax
import jax.numpy as jnp
from jax import lax
from jax.experimental import pallas as pl
from jax.experimental.pallas import tpu as pltpu
from jax.experimental.pallas import tpu_sc as plsc

N = 100000
E = 1600000
G = 256
H = 64
NUM_OUT = 5

K = 512
ET = 53248
WN = ET // K
EP = 32 * ET
NF = H // 16
ND = N + 16

R = 2048
NB = 49
NPAD = NB * R

PW = 3136
PWIN = 112
PNW = PW // PWIN
PG = 272


def _iota16():
    return lax.iota(jnp.int32, 16)


def _dense_body(first, refs):
    i = pl.program_id(0)
    if first:
        (x_ref, d0_ref, d1_ref, bp_ref, w_ref, asrc_ref, adst_ref,
         h_ref, as_ref, ad_ref, mx_ref) = refs
        act = x_ref[...]
    else:
        (a00, a01, a02, a03, a10, a11, a12, a13, d0_ref, d1_ref, bp_ref,
         w_ref, asrc_ref, adst_ref, h_ref, as_ref, ad_ref, mx_ref) = refs
        num = jnp.concatenate(
            [a00[...] + a10[...], a01[...] + a11[...],
             a02[...] + a12[...], a03[...] + a13[...]], axis=1)
        den = (d0_ref[...] + d1_ref[...]).reshape(R, 1)
        act = jnp.maximum(num / den + bp_ref[...], 0.0)
    h = jnp.dot(act, w_ref[...], preferred_element_type=jnp.float32)
    h_ref[...] = h
    asb = jnp.dot(h, asrc_ref[...], preferred_element_type=jnp.float32)[:, 0]
    adb = jnp.dot(h, adst_ref[...], preferred_element_type=jnp.float32)[:, 0]
    as_ref[...] = asb.reshape(1, 1, R)
    ad_ref[...] = adb.reshape(1, 1, R)
    valid = i * R + lax.iota(jnp.int32, R) < N
    bmax = jnp.max(jnp.where(valid, asb, -3.4e38))

    @pl.when(i == 0)
    def _():
        mx_ref[...] = jnp.full((1, 1), -3.4e38, jnp.float32)
    mx_ref[...] = jnp.maximum(mx_ref[...], bmax)


def _dense_layer(first, acc_parts, den_pair, b_prev, W, a_src, a_dst):
    kin = acc_parts[0].shape[1] if first else H
    grid = (NB,)
    out_shapes = (
        jax.ShapeDtypeStruct((N, H), jnp.float32),
        jax.ShapeDtypeStruct((NB, 1, R), jnp.float32),
        jax.ShapeDtypeStruct((NB, 1, R), jnp.float32),
        jax.ShapeDtypeStruct((1, 1), jnp.float32),
    )
    part_specs = ([pl.BlockSpec((R, kin), lambda i: (i, 0))] if first else
                  [pl.BlockSpec((R, 16), lambda i: (i, 0))] * 8)
    in_specs = part_specs + [
        pl.BlockSpec((1, 1, R), lambda i: (i, 0, 0)),
        pl.BlockSpec((1, 1, R), lambda i: (i, 0, 0)),
        pl.BlockSpec((1, H), lambda i: (0, 0)),
        pl.BlockSpec((kin, H), lambda i: (0, 0)),
        pl.BlockSpec((H, 1), lambda i: (0, 0)),
        pl.BlockSpec((H, 1), lambda i: (0, 0)),
    ]
    out_specs = (
        pl.BlockSpec((R, H), lambda i: (i, 0)),
        pl.BlockSpec((1, 1, R), lambda i: (i, 0, 0)),
        pl.BlockSpec((1, 1, R), lambda i: (i, 0, 0)),
        pl.BlockSpec((1, 1), lambda i: (0, 0)),
    )

    def body(*refs):
        _dense_body(first, refs)

    h, as3d, ad3d, mx = pl.pallas_call(
        body,
        grid=grid,
        in_specs=in_specs,
        out_specs=out_specs,
        out_shape=out_shapes,
    )(*acc_parts, den_pair[0], den_pair[1], b_prev.reshape(1, H), W,
      a_src.reshape(H, 1), a_dst.reshape(H, 1))
    as_flat = as3d.reshape(NPAD)[:N]
    ad_flat = ad3d.reshape(NPAD)[:N]
    mx16 = jnp.broadcast_to(mx.reshape(1), (16,))
    return h, as_flat, ad_flat, mx16


def _edge_body(hcat_hbm, as_hbm, ad_hbm, mx_hbm, src_hbm, dst_hbm,
               z16_hbm, z1_hbm,
               acc_hbm, den_hbm, w_hbm,
               srcw, dstw, asv, adv,
               wbuf0, gix0, dli0, hsl0,
               wbuf1, gix1, dli1, hsl1,
               mxv, acc_sh, den_sh,
               semL, sem3, semw, semg0, semg1, sems0, sems1):
    bufs = [(wbuf0, gix0, dli0, hsl0), (wbuf1, gix1, dli1, hsl1)]
    semg = [semg0, semg1]
    sems = [sems0, sems1]
    c = lax.axis_index("c")
    s = lax.axis_index("s")
    w32 = c * 16 + s
    base_e = w32 * ET
    pltpu.sync_copy(mx_hbm, mxv)
    asmax = mxv[pl.ds(0, 16)][0]

    def pass_body(p, _):
        @pl.when(s < 10)
        def _():
            pltpu.sync_copy(z16_hbm, acc_sh.at[pl.ds(s * 10000, 10000), :])

        @pl.when(s == 10)
        def _():
            pltpu.sync_copy(z16_hbm.at[pl.ds(0, 16), :],
                            acc_sh.at[pl.ds(N, 16), :])

        @pl.when(p == 0)
        def _():
            @pl.when(s < 10)
            def _():
                pltpu.sync_copy(z1_hbm, den_sh.at[pl.ds(s * 10000, 10000)])

            @pl.when(s == 10)
            def _():
                pltpu.sync_copy(z1_hbm.at[pl.ds(0, 16)],
                                den_sh.at[pl.ds(N, 16)])
        plsc.subcore_barrier()

        def stage1(eb, bb):
            wbuf, gix, dli, hsl = bufs[bb]
            d1 = pltpu.async_copy(src_hbm.at[pl.ds(eb, K)], srcw, semL)
            d2 = pltpu.async_copy(dst_hbm.at[pl.ds(eb, K)], dstw, semL)
            d1.wait()
            d2.wait()
            for t in range(K // 16):
                sl = pl.ds((t % 8) * 16, 16)
                gix[t // 8, sl] = srcw[pl.ds(t * 16, 16)] + p * N
                dli[t // 8, sl] = dstw[pl.ds(t * 16, 16)]
            hdescs = []
            for j in range(K // 128):
                hdescs.append(pltpu.async_copy(
                    hcat_hbm.at[gix.at[j]],
                    hsl.at[pl.ds(j * 128, 128), :], semg[bb]))

            @pl.when(p == 0)
            def _():
                adescs = []
                for j in range(K // 128):
                    adescs.append(pltpu.async_copy(
                        as_hbm.at[srcw.at[pl.ds(j * 128, 128)]],
                        asv.at[pl.ds(j * 128, 128)], sem3))
                    adescs.append(pltpu.async_copy(
                        ad_hbm.at[dstw.at[pl.ds(j * 128, 128)]],
                        adv.at[pl.ds(j * 128, 128)], sem3))
                for d in adescs:
                    d.wait()
                for t in range(K // 16):
                    sl = pl.ds(t * 16, 16)
                    av = asv[sl]
                    bv = adv[sl]
                    z = av + bv
                    e = jnp.maximum(z, 0.2 * z)
                    u = asmax + bv
                    cb = jnp.maximum(u, 0.2 * u)
                    wbuf[sl] = jnp.exp(e - cb)
                pltpu.async_copy(wbuf, w_hbm.at[pl.ds(eb, K)], semw).wait()

            @pl.when(p != 0)
            def _():
                pltpu.async_copy(w_hbm.at[pl.ds(eb, K)], wbuf, semw).wait()
            return hdescs

        def stage2(bb, hdescs):
            wbuf, gix, dli, hsl = bufs[bb]
            for d in hdescs:
                d.wait()

            def scale_grp(gb, _):
                wv = wbuf[pl.ds(gb * 16, 16)]
                for i in range(16):
                    r = gb * 16 + i
                    hsl[r, pl.ds(0, 16)] = hsl[r, pl.ds(0, 16)] * wv[i]
                return 0
            lax.fori_loop(0, K // 16, scale_grp, 0)

            sdescs = []
            for j in range(K // 128):
                sdescs.append(pltpu.async_copy(
                    hsl.at[pl.ds(j * 128, 128), :],
                    acc_sh.at[dli.at[j]], sems[bb], add=True))

            @pl.when(p == 0)
            def _():
                ddescs = []
                for j in range(K // 128):
                    ddescs.append(pltpu.async_copy(
                        wbuf.at[pl.ds(j * 128, 128)],
                        den_sh.at[dli.at[j]], sems[bb], add=True))
                for d in ddescs:
                    d.wait()
            return sdescs

        def pair_body(pi, _):
            eb = base_e + pi * (2 * K)
            hA = stage1(eb, 0)
            hB = stage1(eb + K, 1)
            sA = stage2(0, hA)
            sB = stage2(1, hB)
            for d in sA:
                d.wait()
            for d in sB:
                d.wait()
            return 0

        lax.fori_loop(0, WN // 2, pair_body, 0)
        plsc.subcore_barrier()

        @pl.when(s < 10)
        def _():
            pltpu.sync_copy(
                acc_sh.at[pl.ds(s * 10000, 10000), :],
                acc_hbm.at[pl.ds((c * NF + p) * N + s * 10000, 10000), :])

            @pl.when(p == 0)
            def _():
                pltpu.sync_copy(
                    den_sh.at[pl.ds(s * 10000, 10000)],
                    den_hbm.at[pl.ds(c * N + s * 10000, 10000)])
        plsc.subcore_barrier()
        return 0

    lax.fori_loop(0, NF, pass_body, 0)


def _make_edge_kernel():
    mesh = plsc.VectorSubcoreMesh(core_axis_name="c", subcore_axis_name="s")
    return pl.kernel(
        _edge_body,
        out_type=(
            jax.ShapeDtypeStruct((2 * NF * N, 16), jnp.float32),
            jax.ShapeDtypeStruct((2 * N,), jnp.float32),
            jax.ShapeDtypeStruct((EP,), jnp.float32),
        ),
        mesh=mesh,
        compiler_params=pltpu.CompilerParams(use_tc_tiling_on_sc=False),
        scratch_types=(
            [pltpu.VMEM((K,), jnp.int32),
             pltpu.VMEM((K,), jnp.int32),
             pltpu.VMEM((K,), jnp.float32),
             pltpu.VMEM((K,), jnp.float32)] +
            [pltpu.VMEM((K,), jnp.float32),
             pltpu.VMEM((K // 128, 128), jnp.int32),
             pltpu.VMEM((K // 128, 128), jnp.int32),
             pltpu.VMEM((K, 16), jnp.float32)] * 2 +
            [pltpu.VMEM((16,), jnp.float32),
             pltpu.VMEM_SHARED((ND, 16), jnp.float32),
             pltpu.VMEM_SHARED((ND,), jnp.float32)] +
            [pltpu.SemaphoreType.DMA] * 7
        ),
    )


def _pool_body(a0_hbm, a1_hbm, d_hbm, b_hbm, batch_hbm, zp_hbm, zc_hbm,
               psum_hbm, pcnt_hbm,
               arow, brow, drow, bidx, ones_v, bvec, psh, csh, psem):
    c = lax.axis_index("c")
    s = lax.axis_index("s")
    w = s * 2 + c
    pltpu.sync_copy(b_hbm, bvec)
    bch = [bvec[pl.ds(j * 16, 16)] for j in range(NF)]
    for t in range(PWIN // 16):
        ones_v[pl.ds(t * 16, 16)] = jnp.ones((16,), jnp.float32)

    @pl.when(s < 2)
    def _():
        pltpu.sync_copy(zp_hbm, psh.at[pl.ds(s * (PG // 2), PG // 2), :])
        pltpu.sync_copy(zc_hbm, csh.at[pl.ds(s * (PG // 2), PG // 2)])
    plsc.subcore_barrier()

    def win_body(wi, _):
        nb = w * PW + wi * PWIN
        descs = []
        for j in range(NF):
            descs.append(pltpu.async_copy(
                a0_hbm.at[pl.ds(j * NPAD + nb, PWIN), :],
                arow.at[pl.ds(0, PWIN), pl.ds(j * 16, 16)], psem))
            descs.append(pltpu.async_copy(
                a1_hbm.at[pl.ds(j * NPAD + nb, PWIN), :],
                brow.at[pl.ds(0, PWIN), pl.ds(j * 16, 16)], psem))
        descs.append(pltpu.async_copy(d_hbm.at[pl.ds(nb, PWIN)],
                                      drow.at[pl.ds(0, PWIN)], psem))
        descs.append(pltpu.async_copy(d_hbm.at[pl.ds(NPAD + nb, PWIN)],
                                      drow.at[pl.ds(PWIN, PWIN)], psem))
        descs.append(pltpu.async_copy(batch_hbm.at[pl.ds(nb, PWIN)],
                                      bidx.at[0], psem))
        for d in descs:
            d.wait()

        def row_grp(t, _):
            dv = (drow[pl.ds(t * 16, 16)] +
                  drow[pl.ds(PWIN + t * 16, 16)])
            inv = 1.0 / dv
            for i in range(16):
                r = t * 16 + i
                for j in range(NF):
                    sl = pl.ds(j * 16, 16)
                    v = arow[r, sl] + brow[r, sl]
                    arow[r, sl] = jnp.maximum(v * inv[i] + bch[j], 0.0)
            return 0
        lax.fori_loop(0, PWIN // 16, row_grp, 0)

        pltpu.sync_copy(arow, psh.at[bidx.at[0]], add=True)
        pltpu.sync_copy(ones_v, csh.at[bidx.at[0]], add=True)
        return 0
    lax.fori_loop(0, PNW, win_body, 0)
    plsc.subcore_barrier()

    @pl.when(s < 2)
    def _():
        pltpu.sync_copy(psh.at[pl.ds(s * (PG // 2), PG // 2), :],
                        psum_hbm.at[pl.ds(c * PG + s * (PG // 2), PG // 2), :])
        pltpu.sync_copy(csh.at[pl.ds(s * (PG // 2), PG // 2)],
                        pcnt_hbm.at[pl.ds(c * PG + s * (PG // 2), PG // 2)])


def _make_pool_kernel():
    mesh = plsc.VectorSubcoreMesh(core_axis_name="c", subcore_axis_name="s")
    return pl.kernel(
        _pool_body,
        out_type=(
            jax.ShapeDtypeStruct((2 * PG, H), jnp.float32),
            jax.ShapeDtypeStruct((2 * PG,), jnp.float32),
        ),
        mesh=mesh,
        compiler_params=pltpu.CompilerParams(use_tc_tiling_on_sc=False),
        scratch_types=[
            pltpu.VMEM((PWIN, H), jnp.float32),
            pltpu.VMEM((PWIN, H), jnp.float32),
            pltpu.VMEM((2 * PWIN,), jnp.float32),
            pltpu.VMEM((1, PWIN), jnp.int32),
            pltpu.VMEM((PWIN,), jnp.float32),
            pltpu.VMEM((H,), jnp.float32),
            pltpu.VMEM_SHARED((PG, H), jnp.float32),
            pltpu.VMEM_SHARED((PG,), jnp.float32),
            pltpu.SemaphoreType.DMA,
        ],
    )


def _head_body(ps_ref, pc_ref, wp_ref, bp_ref, o_ref):
    sums = ps_ref[0, :G, :] + ps_ref[1, :G, :]
    cnts = pc_ref[0, 0, :G] + pc_ref[1, 0, :G]
    pooled = sums / jnp.maximum(cnts, 1.0)[:, None]
    o_ref[...] = jnp.dot(pooled, wp_ref[...],
                         preferred_element_type=jnp.float32) + bp_ref[...]


def _split_parts(accp, denp):
    parts = [accp[(c * NF + j) * N:(c * NF + j + 1) * N]
             for c in range(2) for j in range(NF)]
    den0 = jnp.pad(denp[:N], (0, NPAD - N),
                   constant_values=1.0).reshape(NB, 1, R)
    den1 = jnp.pad(denp[N:], (0, NPAD - N),
                   constant_values=1.0).reshape(NB, 1, R)
    return parts, (den0, den1)


def kernel(x, edge_index, batch, W1, a_src1, a_dst1, b1, W2, a_src2, a_dst2,
           b2, W3, a_src3, a_dst3, b3, Wp, bp):
    f32 = jnp.float32
    loop = jnp.arange(N, dtype=edge_index.dtype)
    pad_e = EP - (E + N)
    srcp = jnp.concatenate([edge_index[0], loop,
                            jnp.zeros((pad_e,), jnp.int32)])
    dstp = jnp.concatenate([edge_index[1], loop,
                            jnp.full((pad_e,), N, jnp.int32)])

    x_pad = jnp.pad(x, ((0, 0), (0, 5)))
    W1p = jnp.pad(W1, ((0, 5), (0, 0)))

    z16 = jnp.zeros((10000, 16), f32)
    z1 = jnp.zeros((10000,), f32)
    dummy_den = jnp.ones((NB, 1, R), f32)

    edge_k = _make_edge_kernel()

    def run_layer(first, parts, dens, b_prev, W, a_src, a_dst):
        h, as_, ad_, mx = _dense_layer(first, parts, dens, b_prev, W,
                                       a_src, a_dst)
        hcat = jnp.concatenate([h[:, j * 16:(j + 1) * 16]
                                for j in range(NF)], axis=0)
        ad_pad = jnp.pad(ad_, (0, 16))
        accp, denp, _ = edge_k(hcat, as_, ad_pad, mx, srcp, dstp, z16, z1)
        return _split_parts(accp, denp)

    parts1, dens1 = run_layer(True, [x_pad], (dummy_den, dummy_den),
                              b1, W1p, a_src1, a_dst1)
    parts2, dens2 = run_layer(False, parts1, dens1, b1, W2, a_src2, a_dst2)
    parts3, dens3 = run_layer(False, parts2, dens2, b2, W3, a_src3, a_dst3)

    a0 = jnp.concatenate(parts3[:NF], axis=0)
    a1 = jnp.concatenate(parts3[NF:], axis=0)
    a0 = jnp.pad(a0.reshape(NF, N, 16), ((0, 0), (0, NPAD - N), (0, 0))
                 ).reshape(NF * NPAD, 16)
    a1 = jnp.pad(a1.reshape(NF, N, 16), ((0, 0), (0, NPAD - N), (0, 0))
                 ).reshape(NF * NPAD, 16)
    dnp = jnp.concatenate([
        jnp.pad(dens3[0].reshape(NPAD)[:N], (0, NPAD - N), constant_values=1.0),
        jnp.pad(dens3[1].reshape(NPAD)[:N], (0, NPAD - N), constant_values=1.0),
    ])
    batchp = jnp.pad(batch, (0, NPAD - N), constant_values=G)
    zp = jnp.zeros((PG // 2, H), f32)
    zc = jnp.zeros((PG // 2,), f32)
    pool_k = _make_pool_kernel()
    psum, pcnt = pool_k(a0, a1, dnp, b3, batchp, zp, zc)

    Wpp = jnp.zeros((H, 128), f32).at[:, :NUM_OUT].set(Wp)
    bpp = jnp.zeros((1, 128), f32).at[0, :NUM_OUT].set(bp)
    out_pad = pl.pallas_call(
        _head_body,
        out_shape=jax.ShapeDtypeStruct((G, 128), f32),
    )(psum.reshape(2, PG, H), pcnt.reshape(2, 1, PG), Wpp, bpp)
    return out_pad[:, :NUM_OUT]


# speedup vs baseline: 23.5641x; 1.0220x over previous
import functools

import jax
import jax.numpy as jnp
from jax import lax
from jax.experimental import pallas as pl
from jax.experimental.pallas import tpu as pltpu
from jax.experimental.pallas import tpu_sc as plsc

N = 100000
E = 1600000
G = 256
H = 64
NUM_OUT = 5

K = 512
ET = 53248
WN = ET // K
EP = 32 * ET
NF = H // 16
ND = N + 16

R = 2048
NB = 49
NPAD = NB * R

PW = 3136
PWIN = 112
PNW = PW // PWIN
PG = 272


def _iota16():
    return lax.iota(jnp.int32, 16)


def _dense_body(first, refs):
    i = pl.program_id(0)
    if first:
        (x_ref, d0_ref, d1_ref, bp_ref, w_ref, asrc_ref, adst_ref,
         h_ref, as_ref, ad_ref, mx_ref) = refs
        act = x_ref[...]
    else:
        (a00, a01, a02, a03, a10, a11, a12, a13, d0_ref, d1_ref, bp_ref,
         w_ref, asrc_ref, adst_ref, h_ref, as_ref, ad_ref, mx_ref) = refs
        num = jnp.concatenate(
            [a00[...] + a10[...], a01[...] + a11[...],
             a02[...] + a12[...], a03[...] + a13[...]], axis=1)
        den = (d0_ref[...] + d1_ref[...]).reshape(R, 1)
        act = jnp.maximum(num / den + bp_ref[...], 0.0)
    h = jnp.dot(act, w_ref[...], preferred_element_type=jnp.float32)
    h_ref[...] = h
    asb = jnp.dot(h, asrc_ref[...], preferred_element_type=jnp.float32)[:, 0]
    adb = jnp.dot(h, adst_ref[...], preferred_element_type=jnp.float32)[:, 0]
    as_ref[...] = asb.reshape(1, 1, R)
    ad_ref[...] = adb.reshape(1, 1, R)
    valid = i * R + lax.iota(jnp.int32, R) < N
    bmax = jnp.max(jnp.where(valid, asb, -3.4e38))

    @pl.when(i == 0)
    def _():
        mx_ref[...] = jnp.full((1, 1), -3.4e38, jnp.float32)
    mx_ref[...] = jnp.maximum(mx_ref[...], bmax)


def _dense_layer(first, acc_parts, den_pair, b_prev, W, a_src, a_dst):
    kin = acc_parts[0].shape[1] if first else H
    grid = (NB,)
    out_shapes = (
        jax.ShapeDtypeStruct((N, H), jnp.float32),
        jax.ShapeDtypeStruct((NB, 1, R), jnp.float32),
        jax.ShapeDtypeStruct((NB, 1, R), jnp.float32),
        jax.ShapeDtypeStruct((1, 1), jnp.float32),
    )
    part_specs = ([pl.BlockSpec((R, kin), lambda i: (i, 0))] if first else
                  [pl.BlockSpec((R, 16), lambda i: (i, 0))] * 8)
    in_specs = part_specs + [
        pl.BlockSpec((1, 1, R), lambda i: (i, 0, 0)),
        pl.BlockSpec((1, 1, R), lambda i: (i, 0, 0)),
        pl.BlockSpec((1, H), lambda i: (0, 0)),
        pl.BlockSpec((kin, H), lambda i: (0, 0)),
        pl.BlockSpec((H, 1), lambda i: (0, 0)),
        pl.BlockSpec((H, 1), lambda i: (0, 0)),
    ]
    out_specs = (
        pl.BlockSpec((R, H), lambda i: (i, 0)),
        pl.BlockSpec((1, 1, R), lambda i: (i, 0, 0)),
        pl.BlockSpec((1, 1, R), lambda i: (i, 0, 0)),
        pl.BlockSpec((1, 1), lambda i: (0, 0)),
    )

    def body(*refs):
        _dense_body(first, refs)

    h, as3d, ad3d, mx = pl.pallas_call(
        body,
        grid=grid,
        in_specs=in_specs,
        out_specs=out_specs,
        out_shape=out_shapes,
    )(*acc_parts, den_pair[0], den_pair[1], b_prev.reshape(1, H), W,
      a_src.reshape(H, 1), a_dst.reshape(H, 1))
    as_flat = as3d.reshape(NPAD)[:N]
    ad_flat = ad3d.reshape(NPAD)[:N]
    mx16 = jnp.broadcast_to(mx.reshape(1), (16,))
    return h, as_flat, ad_flat, mx16


def _edge_body(hcat_hbm, as_hbm, ad_hbm, mx_hbm, src_hbm, dst_hbm,
               z16_hbm, z1_hbm,
               acc_hbm, den_hbm, w_hbm,
               srcw, dstw, asv, adv,
               wbuf0, gix0, dli0, hsl0,
               wbuf1, gix1, dli1, hsl1,
               mxv, acc_sh, den_sh,
               semL, sem3, semw, semg0, semg1, sems0, sems1):
    bufs = [(wbuf0, gix0, dli0, hsl0), (wbuf1, gix1, dli1, hsl1)]
    semg = [semg0, semg1]
    sems = [sems0, sems1]
    c = lax.axis_index("c")
    s = lax.axis_index("s")
    w32 = c * 16 + s
    base_e = w32 * ET
    pltpu.sync_copy(mx_hbm, mxv)
    asmax = mxv[pl.ds(0, 16)][0]

    def pass_body(p, _):
        @pl.when(s < 10)
        def _():
            pltpu.sync_copy(z16_hbm, acc_sh.at[pl.ds(s * 10000, 10000), :])

        @pl.when(s == 10)
        def _():
            pltpu.sync_copy(z16_hbm.at[pl.ds(0, 16), :],
                            acc_sh.at[pl.ds(N, 16), :])

        @pl.when(p == 0)
        def _():
            @pl.when(s < 10)
            def _():
                pltpu.sync_copy(z1_hbm, den_sh.at[pl.ds(s * 10000, 10000)])

            @pl.when(s == 10)
            def _():
                pltpu.sync_copy(z1_hbm.at[pl.ds(0, 16)],
                                den_sh.at[pl.ds(N, 16)])
        plsc.subcore_barrier()

        def stage1(eb, bb):
            wbuf, gix, dli, hsl = bufs[bb]
            d1 = pltpu.async_copy(src_hbm.at[pl.ds(eb, K)], srcw, semL)
            d2 = pltpu.async_copy(dst_hbm.at[pl.ds(eb, K)], dstw, semL)

            @pl.when(p != 0)
            def _():
                pltpu.async_copy(w_hbm.at[pl.ds(eb, K)], wbuf, semw).wait()
            d1.wait()
            d2.wait()
            for t in range(K // 16):
                sl = pl.ds((t % 8) * 16, 16)
                gix[t // 8, sl] = srcw[pl.ds(t * 16, 16)] + p * N
                dli[t // 8, sl] = dstw[pl.ds(t * 16, 16)]
            hdescs = []
            for j in range(K // 128):
                hdescs.append(pltpu.async_copy(
                    hcat_hbm.at[gix.at[j]],
                    hsl.at[pl.ds(j * 128, 128), :], semg[bb]))

            @pl.when(p == 0)
            def _():
                adescs = []
                for j in range(K // 128):
                    adescs.append(pltpu.async_copy(
                        as_hbm.at[srcw.at[pl.ds(j * 128, 128)]],
                        asv.at[pl.ds(j * 128, 128)], sem3))
                    adescs.append(pltpu.async_copy(
                        ad_hbm.at[dstw.at[pl.ds(j * 128, 128)]],
                        adv.at[pl.ds(j * 128, 128)], sem3))
                for d in adescs:
                    d.wait()
                for t in range(K // 16):
                    sl = pl.ds(t * 16, 16)
                    av = asv[sl]
                    bv = adv[sl]
                    z = av + bv
                    e = jnp.maximum(z, 0.2 * z)
                    u = asmax + bv
                    cb = jnp.maximum(u, 0.2 * u)
                    wbuf[sl] = jnp.exp(e - cb)
                pltpu.async_copy(wbuf, w_hbm.at[pl.ds(eb, K)], semw).wait()
            return hdescs

        def stage2(bb, hdescs):
            wbuf, gix, dli, hsl = bufs[bb]
            for d in hdescs:
                d.wait()

            def scale_grp(gb, _):
                wv = wbuf[pl.ds(gb * 16, 16)]
                for i in range(16):
                    r = gb * 16 + i
                    hsl[r, pl.ds(0, 16)] = hsl[r, pl.ds(0, 16)] * wv[i]
                return 0
            lax.fori_loop(0, K // 16, scale_grp, 0)

            sdescs = []
            for j in range(K // 128):
                sdescs.append(pltpu.async_copy(
                    hsl.at[pl.ds(j * 128, 128), :],
                    acc_sh.at[dli.at[j]], sems[bb], add=True))

            @pl.when(p == 0)
            def _():
                ddescs = []
                for j in range(K // 128):
                    ddescs.append(pltpu.async_copy(
                        wbuf.at[pl.ds(j * 128, 128)],
                        den_sh.at[dli.at[j]], sems[bb], add=True))
                for d in ddescs:
                    d.wait()
            return sdescs

        def pair_body(pi, _):
            eb = base_e + pi * (2 * K)
            hA = stage1(eb, 0)
            hB = stage1(eb + K, 1)
            sA = stage2(0, hA)
            sB = stage2(1, hB)
            for d in sA:
                d.wait()
            for d in sB:
                d.wait()
            return 0

        lax.fori_loop(0, WN // 2, pair_body, 0)
        plsc.subcore_barrier()

        @pl.when(s < 10)
        def _():
            pltpu.sync_copy(
                acc_sh.at[pl.ds(s * 10000, 10000), :],
                acc_hbm.at[pl.ds((c * NF + p) * N + s * 10000, 10000), :])

            @pl.when(p == 0)
            def _():
                pltpu.sync_copy(
                    den_sh.at[pl.ds(s * 10000, 10000)],
                    den_hbm.at[pl.ds(c * N + s * 10000, 10000)])
        plsc.subcore_barrier()
        return 0

    lax.fori_loop(0, NF, pass_body, 0)


def _make_edge_kernel():
    mesh = plsc.VectorSubcoreMesh(core_axis_name="c", subcore_axis_name="s")
    return pl.kernel(
        _edge_body,
        out_type=(
            jax.ShapeDtypeStruct((2 * NF * N, 16), jnp.float32),
            jax.ShapeDtypeStruct((2 * N,), jnp.float32),
            jax.ShapeDtypeStruct((EP,), jnp.float32),
        ),
        mesh=mesh,
        compiler_params=pltpu.CompilerParams(use_tc_tiling_on_sc=False),
        scratch_types=(
            [pltpu.VMEM((K,), jnp.int32),
             pltpu.VMEM((K,), jnp.int32),
             pltpu.VMEM((K,), jnp.float32),
             pltpu.VMEM((K,), jnp.float32)] +
            [pltpu.VMEM((K,), jnp.float32),
             pltpu.VMEM((K // 128, 128), jnp.int32),
             pltpu.VMEM((K // 128, 128), jnp.int32),
             pltpu.VMEM((K, 16), jnp.float32)] * 2 +
            [pltpu.VMEM((16,), jnp.float32),
             pltpu.VMEM_SHARED((ND, 16), jnp.float32),
             pltpu.VMEM_SHARED((ND,), jnp.float32)] +
            [pltpu.SemaphoreType.DMA] * 7
        ),
    )


def _pool_body(a0_hbm, a1_hbm, d_hbm, b_hbm, batch_hbm, zp_hbm, zc_hbm,
               psum_hbm, pcnt_hbm,
               arow, brow, drow, bidx, ones_v, bvec, psh, csh, psem):
    c = lax.axis_index("c")
    s = lax.axis_index("s")
    w = s * 2 + c
    pltpu.sync_copy(b_hbm, bvec)
    bch = [bvec[pl.ds(j * 16, 16)] for j in range(NF)]
    for t in range(PWIN // 16):
        ones_v[pl.ds(t * 16, 16)] = jnp.ones((16,), jnp.float32)

    @pl.when(s < 2)
    def _():
        pltpu.sync_copy(zp_hbm, psh.at[pl.ds(s * (PG // 2), PG // 2), :])
        pltpu.sync_copy(zc_hbm, csh.at[pl.ds(s * (PG // 2), PG // 2)])
    plsc.subcore_barrier()

    def win_body(wi, _):
        nb = w * PW + wi * PWIN
        descs = []
        for j in range(NF):
            descs.append(pltpu.async_copy(
                a0_hbm.at[pl.ds(j * NPAD + nb, PWIN), :],
                arow.at[pl.ds(0, PWIN), pl.ds(j * 16, 16)], psem))
            descs.append(pltpu.async_copy(
                a1_hbm.at[pl.ds(j * NPAD + nb, PWIN), :],
                brow.at[pl.ds(0, PWIN), pl.ds(j * 16, 16)], psem))
        descs.append(pltpu.async_copy(d_hbm.at[pl.ds(nb, PWIN)],
                                      drow.at[pl.ds(0, PWIN)], psem))
        descs.append(pltpu.async_copy(d_hbm.at[pl.ds(NPAD + nb, PWIN)],
                                      drow.at[pl.ds(PWIN, PWIN)], psem))
        descs.append(pltpu.async_copy(batch_hbm.at[pl.ds(nb, PWIN)],
                                      bidx.at[0], psem))
        for d in descs:
            d.wait()

        def row_grp(t, _):
            dv = (drow[pl.ds(t * 16, 16)] +
                  drow[pl.ds(PWIN + t * 16, 16)])
            inv = 1.0 / dv
            for i in range(16):
                r = t * 16 + i
                for j in range(NF):
                    sl = pl.ds(j * 16, 16)
                    v = arow[r, sl] + brow[r, sl]
                    arow[r, sl] = jnp.maximum(v * inv[i] + bch[j], 0.0)
            return 0
        lax.fori_loop(0, PWIN // 16, row_grp, 0)

        pltpu.sync_copy(arow, psh.at[bidx.at[0]], add=True)
        pltpu.sync_copy(ones_v, csh.at[bidx.at[0]], add=True)
        return 0
    lax.fori_loop(0, PNW, win_body, 0)
    plsc.subcore_barrier()

    @pl.when(s < 2)
    def _():
        pltpu.sync_copy(psh.at[pl.ds(s * (PG // 2), PG // 2), :],
                        psum_hbm.at[pl.ds(c * PG + s * (PG // 2), PG // 2), :])
        pltpu.sync_copy(csh.at[pl.ds(s * (PG // 2), PG // 2)],
                        pcnt_hbm.at[pl.ds(c * PG + s * (PG // 2), PG // 2)])


def _make_pool_kernel():
    mesh = plsc.VectorSubcoreMesh(core_axis_name="c", subcore_axis_name="s")
    return pl.kernel(
        _pool_body,
        out_type=(
            jax.ShapeDtypeStruct((2 * PG, H), jnp.float32),
            jax.ShapeDtypeStruct((2 * PG,), jnp.float32),
        ),
        mesh=mesh,
        compiler_params=pltpu.CompilerParams(use_tc_tiling_on_sc=False),
        scratch_types=[
            pltpu.VMEM((PWIN, H), jnp.float32),
            pltpu.VMEM((PWIN, H), jnp.float32),
            pltpu.VMEM((2 * PWIN,), jnp.float32),
            pltpu.VMEM((1, PWIN), jnp.int32),
            pltpu.VMEM((PWIN,), jnp.float32),
            pltpu.VMEM((H,), jnp.float32),
            pltpu.VMEM_SHARED((PG, H), jnp.float32),
            pltpu.VMEM_SHARED((PG,), jnp.float32),
            pltpu.SemaphoreType.DMA,
        ],
    )


def _head_body(ps_ref, pc_ref, wp_ref, bp_ref, o_ref):
    sums = ps_ref[0, :G, :] + ps_ref[1, :G, :]
    cnts = pc_ref[0, 0, :G] + pc_ref[1, 0, :G]
    pooled = sums / jnp.maximum(cnts, 1.0)[:, None]
    o_ref[...] = jnp.dot(pooled, wp_ref[...],
                         preferred_element_type=jnp.float32) + bp_ref[...]


def _split_parts(accp, denp):
    parts = [accp[(c * NF + j) * N:(c * NF + j + 1) * N]
             for c in range(2) for j in range(NF)]
    den0 = jnp.pad(denp[:N], (0, NPAD - N),
                   constant_values=1.0).reshape(NB, 1, R)
    den1 = jnp.pad(denp[N:], (0, NPAD - N),
                   constant_values=1.0).reshape(NB, 1, R)
    return parts, (den0, den1)


def kernel(x, edge_index, batch, W1, a_src1, a_dst1, b1, W2, a_src2, a_dst2,
           b2, W3, a_src3, a_dst3, b3, Wp, bp):
    f32 = jnp.float32
    loop = jnp.arange(N, dtype=edge_index.dtype)
    pad_e = EP - (E + N)
    srcp = jnp.concatenate([edge_index[0], loop,
                            jnp.zeros((pad_e,), jnp.int32)])
    dstp = jnp.concatenate([edge_index[1], loop,
                            jnp.full((pad_e,), N, jnp.int32)])

    x_pad = jnp.pad(x, ((0, 0), (0, 5)))
    W1p = jnp.pad(W1, ((0, 5), (0, 0)))

    z16 = jnp.zeros((10000, 16), f32)
    z1 = jnp.zeros((10000,), f32)
    dummy_den = jnp.ones((NB, 1, R), f32)

    edge_k = _make_edge_kernel()

    def run_layer(first, parts, dens, b_prev, W, a_src, a_dst):
        h, as_, ad_, mx = _dense_layer(first, parts, dens, b_prev, W,
                                       a_src, a_dst)
        hcat = jnp.concatenate([h[:, j * 16:(j + 1) * 16]
                                for j in range(NF)], axis=0)
        ad_pad = jnp.pad(ad_, (0, 16))
        accp, denp, _ = edge_k(hcat, as_, ad_pad, mx, srcp, dstp, z16, z1)
        return _split_parts(accp, denp)

    parts1, dens1 = run_layer(True, [x_pad], (dummy_den, dummy_den),
                              b1, W1p, a_src1, a_dst1)
    parts2, dens2 = run_layer(False, parts1, dens1, b1, W2, a_src2, a_dst2)
    parts3, dens3 = run_layer(False, parts2, dens2, b2, W3, a_src3, a_dst3)

    a0 = jnp.concatenate(parts3[:NF], axis=0)
    a1 = jnp.concatenate(parts3[NF:], axis=0)
    a0 = jnp.pad(a0.reshape(NF, N, 16), ((0, 0), (0, NPAD - N), (0, 0))
                 ).reshape(NF * NPAD, 16)
    a1 = jnp.pad(a1.reshape(NF, N, 16), ((0, 0), (0, NPAD - N), (0, 0))
                 ).reshape(NF * NPAD, 16)
    dnp = jnp.concatenate([
        jnp.pad(dens3[0].reshape(NPAD)[:N], (0, NPAD - N), constant_values=1.0),
        jnp.pad(dens3[1].reshape(NPAD)[:N], (0, NPAD - N), constant_values=1.0),
    ])
    batchp = jnp.pad(batch, (0, NPAD - N), constant_values=G)
    zp = jnp.zeros((PG // 2, H), f32)
    zc = jnp.zeros((PG // 2,), f32)
    pool_k = _make_pool_kernel()
    psum, pcnt = pool_k(a0, a1, dnp, b3, batchp, zp, zc)

    Wpp = jnp.zeros((H, 128), f32).at[:, :NUM_OUT].set(Wp)
    bpp = jnp.zeros((1, 128), f32).at[0, :NUM_OUT].set(bp)
    out_pad = pl.pallas_call(
        _head_body,
        out_shape=jax.ShapeDtypeStruct((G, 128), f32),
    )(psum.reshape(2, PG, H), pcnt.reshape(2, 1, PG), Wpp, bpp)
    return out_pad[:, :NUM_OUT]
